# Initial kernel scaffold; baseline (speedup 1.0000x reference)
import functools

import jax
import jax.numpy as jnp
from jax import lax
from jax.experimental import pallas as pl
from jax.experimental.pallas import tpu as pltpu
from jax.experimental.pallas import tpu_sc as plsc

NN = 10000
DF = 128
DE = 16
NEDGE = 320000
NPOS = 100000
NNEG = 100000
NSIDE = 102400
NLINK = 2 * NSIDE
NPAD = NSIDE - NPOS

NC = 2
NS = 16
NW = NC * NS

MP_K = 80
MP_CHUNKS = NEDGE // (NW * MP_K)
MP_ROWS_PER_TILE = NN // NS

LN_K = 128
LN_CHUNKS = NLINK // (NW * LN_K)
LN_PER_W = NLINK // NW

SCORE_R = 512
SCORE_BLOCKS = NLINK // SCORE_R
SIDE_BLOCKS = NSIDE // SCORE_R


def _prep_body(x_ref, xc_ref):
    xv = x_ref[...]
    xc_ref[...] = xv - jnp.mean(xv, axis=0, keepdims=True)


_prep = pl.pallas_call(
    _prep_body,
    out_shape=jax.ShapeDtypeStruct((NN, DF), jnp.float32),
)


def _mp_body(xc_hbm, src_hbm, dst_hbm, zero_hbm, out_hbm,
             src_idx, dst_idx, rows, agg_sh):
    cid = lax.axis_index("c")
    sid = lax.axis_index("s")
    wid = sid * NC + cid
    pltpu.sync_copy(zero_hbm.at[pl.ds(sid * MP_ROWS_PER_TILE, MP_ROWS_PER_TILE)],
                    agg_sh.at[pl.ds(sid * MP_ROWS_PER_TILE, MP_ROWS_PER_TILE)])
    pltpu.sync_copy(src_hbm.at[pl.ds(wid * MP_CHUNKS, MP_CHUNKS)], src_idx)
    pltpu.sync_copy(dst_hbm.at[pl.ds(wid * MP_CHUNKS, MP_CHUNKS)], dst_idx)
    plsc.subcore_barrier()

    def body(j, c):
        pltpu.sync_copy(xc_hbm.at[dst_idx.at[j]], rows)
        pltpu.sync_copy(rows, agg_sh.at[src_idx.at[j]], add=True)
        return c

    lax.fori_loop(0, MP_CHUNKS, body, 0)
    plsc.subcore_barrier()
    base = cid * NN + sid * MP_ROWS_PER_TILE
    pltpu.sync_copy(agg_sh.at[pl.ds(sid * MP_ROWS_PER_TILE, MP_ROWS_PER_TILE)],
                    out_hbm.at[pl.ds(base, MP_ROWS_PER_TILE)])


_mp_call = functools.partial(
    pl.kernel,
    mesh=plsc.VectorSubcoreMesh(core_axis_name="c", subcore_axis_name="s"),
    out_type=jax.ShapeDtypeStruct((NC * NN, DF), jnp.float32),
    scratch_types=[
        pltpu.VMEM((MP_CHUNKS, MP_K), jnp.int32),
        pltpu.VMEM((MP_CHUNKS, MP_K), jnp.int32),
        pltpu.VMEM((MP_K, DF), jnp.float32),
        pltpu.VMEM_SHARED((NN, DF), jnp.float32),
    ],
)(_mp_body)


def _node_body(p_ref, w_ref, h_ref):
    agg = p_ref[0] + p_ref[1]
    h_ref[...] = jnp.maximum(
        jnp.dot(agg, w_ref[...], preferred_element_type=jnp.float32), 0.0)


_node = pl.pallas_call(
    _node_body,
    grid=(10,),
    in_specs=[
        pl.BlockSpec((NC, NN // 10, DF), lambda i: (0, i, 0)),
        pl.BlockSpec((DF, DF), lambda i: (0, 0)),
    ],
    out_specs=pl.BlockSpec((NN // 10, DF), lambda i: (i, 0)),
    out_shape=jax.ShapeDtypeStruct((NN, DF), jnp.float32),
)


def _ln_body(h_hbm, x_hbm, u_hbm, v_hbm, ohu, ohv, oxu, oxv,
             u_idx, v_idx, bhu, bhv, bxu, bxv):
    cid = lax.axis_index("c")
    sid = lax.axis_index("s")
    wid = sid * NC + cid
    pltpu.sync_copy(u_hbm.at[pl.ds(wid * LN_CHUNKS, LN_CHUNKS)], u_idx)
    pltpu.sync_copy(v_hbm.at[pl.ds(wid * LN_CHUNKS, LN_CHUNKS)], v_idx)

    def body(j, c):
        base = wid * LN_PER_W + j * LN_K
        pltpu.sync_copy(h_hbm.at[u_idx.at[j]], bhu)
        pltpu.sync_copy(bhu, ohu.at[pl.ds(base, LN_K)])
        pltpu.sync_copy(h_hbm.at[v_idx.at[j]], bhv)
        pltpu.sync_copy(bhv, ohv.at[pl.ds(base, LN_K)])
        pltpu.sync_copy(x_hbm.at[u_idx.at[j]], bxu)
        pltpu.sync_copy(bxu, oxu.at[pl.ds(base, LN_K)])
        pltpu.sync_copy(x_hbm.at[v_idx.at[j]], bxv)
        pltpu.sync_copy(bxv, oxv.at[pl.ds(base, LN_K)])
        return c

    lax.fori_loop(0, LN_CHUNKS, body, 0)


_gathered_sds = jax.ShapeDtypeStruct((NLINK, DF), jnp.float32)
_ln_call = functools.partial(
    pl.kernel,
    mesh=plsc.VectorSubcoreMesh(core_axis_name="c", subcore_axis_name="s"),
    out_type=(_gathered_sds, _gathered_sds, _gathered_sds, _gathered_sds),
    scratch_types=[
        pltpu.VMEM((LN_CHUNKS, LN_K), jnp.int32),
        pltpu.VMEM((LN_CHUNKS, LN_K), jnp.int32),
        pltpu.VMEM((LN_K, DF), jnp.float32),
        pltpu.VMEM((LN_K, DF), jnp.float32),
        pltpu.VMEM((LN_K, DF), jnp.float32),
        pltpu.VMEM((LN_K, DF), jnp.float32),
    ],
)(_ln_body)


def _score_body(hu, hv, xu, xv, ef, wsn, wse, wg, out):
    i = pl.program_id(0)
    huv = hu[...] * hv[...]
    xd = jnp.abs(xu[...] - xv[...])
    experts = (jnp.dot(huv, wsn[...], preferred_element_type=jnp.float32)
               + jnp.dot(ef[...], wse[...], preferred_element_type=jnp.float32))
    gl = jnp.dot(xd, wg[...], preferred_element_type=jnp.float32)
    col = lax.broadcasted_iota(jnp.int32, (SCORE_R, 8), 1)
    gl = jnp.where(col < 4, gl, -1e30)
    m = jnp.max(gl, axis=1, keepdims=True)
    p = jnp.exp(gl - m)
    gate = p / jnp.sum(p, axis=1, keepdims=True)
    logits = jnp.sum(experts * gate, axis=1, keepdims=True)
    r = lax.broadcasted_iota(jnp.int32, (SCORE_R, 1), 0)
    side_row = i * SCORE_R + r - jnp.where(i < SIDE_BLOCKS, 0, NSIDE)
    valid = side_row < NPOS
    tgt = jnp.where(i < SIDE_BLOCKS, 1.0, 0.0)
    bce = (jnp.maximum(logits, 0.0) - logits * tgt
           + jnp.log(1.0 + jnp.exp(-jnp.abs(logits))))
    s = jnp.sum(jnp.where(valid, bce, 0.0))

    @pl.when(i == 0)
    def _():
        out[0, 0] = s

    @pl.when(i != 0)
    def _():
        out[0, 0] += s


_score = pl.pallas_call(
    _score_body,
    grid=(SCORE_BLOCKS,),
    in_specs=[
        pl.BlockSpec((SCORE_R, DF), lambda i: (i, 0)),
        pl.BlockSpec((SCORE_R, DF), lambda i: (i, 0)),
        pl.BlockSpec((SCORE_R, DF), lambda i: (i, 0)),
        pl.BlockSpec((SCORE_R, DF), lambda i: (i, 0)),
        pl.BlockSpec((SCORE_R, DE), lambda i: (i, 0)),
        pl.BlockSpec((DF, 8), lambda i: (0, 0)),
        pl.BlockSpec((DE, 8), lambda i: (0, 0)),
        pl.BlockSpec((DF, 8), lambda i: (0, 0)),
    ],
    out_specs=pl.BlockSpec((1, 1), lambda i: (0, 0)),
    out_shape=jax.ShapeDtypeStruct((1, 1), jnp.float32),
)


def kernel(x, mp_link, pos_link, neg_link, pos_feats, neg_feats,
           W1, W_score, W_gate):
    f32 = jnp.float32
    msrc = mp_link[:, 0].reshape(NEDGE // MP_K, MP_K)
    mdst = mp_link[:, 1].reshape(NEDGE // MP_K, MP_K)
    pad_idx = (jnp.arange(NPAD, dtype=jnp.int32) * 13) % NN
    u_all = jnp.concatenate(
        [pos_link[:, 0], pad_idx, neg_link[:, 0], pad_idx]).reshape(
            NLINK // LN_K, LN_K)
    v_all = jnp.concatenate(
        [pos_link[:, 1], pad_idx, neg_link[:, 1], pad_idx]).reshape(
            NLINK // LN_K, LN_K)
    zpad = jnp.zeros((NPAD, DE), f32)
    ef = jnp.concatenate([pos_feats, zpad, neg_feats, zpad], axis=0)
    wsn = jnp.pad(W_score[:DF], ((0, 0), (0, 4)))
    wse = jnp.pad(W_score[DF:], ((0, 0), (0, 4)))
    wg = jnp.pad(W_gate, ((0, 0), (0, 4)))
    ztbl = jnp.zeros((NN, DF), f32)

    xc = _prep(x)
    partials = _mp_call(xc, msrc, mdst, ztbl)
    h = _node(partials.reshape(NC, NN, DF), W1)
    hu, hv, xu, xv = _ln_call(h, x, u_all, v_all)
    s = _score(hu, hv, xu, xv, ef, wsn, wse, wg)
    return s[0, 0] * (1.0 / (NPOS + NNEG))


# speedup vs baseline: 2.5489x; 2.5489x over previous
import functools

import jax
import jax.numpy as jnp
from jax import lax
from jax.experimental import pallas as pl
from jax.experimental.pallas import tpu as pltpu
from jax.experimental.pallas import tpu_sc as plsc

NN = 10000
DF = 128
DE = 16
NEDGE = 320000
NPOS = 100000
NNEG = 100000
NSIDE = 102400
NLINK = 2 * NSIDE
NPAD = NSIDE - NPOS

NC = 2
NS = 16
NW = NC * NS

MP_K = 125
MP_CHUNKS = NEDGE // (NW * MP_K)
NNP = 10240
MP_ROWS_PER_TILE = NNP // NS

LN_K = 80
LN_CHUNKS = NLINK // (NW * LN_K)
LN_PER_W = NLINK // NW

SCORE_R = 512
SCORE_BLOCKS = NLINK // SCORE_R
SIDE_BLOCKS = NSIDE // SCORE_R


def _prep_body(x_ref, xc_ref):
    xv = x_ref[...]
    xc_ref[...] = xv - jnp.mean(xv, axis=0, keepdims=True)


_prep = pl.pallas_call(
    _prep_body,
    out_shape=jax.ShapeDtypeStruct((NN, DF), jnp.float32),
)


def _mp_body(xc_hbm, src_hbm, dst_hbm, zero_hbm, out_hbm,
             src_idx, dst_idx, rows, agg_sh):
    cid = lax.axis_index("c")
    sid = lax.axis_index("s")
    wid = sid * NC + cid
    pltpu.sync_copy(zero_hbm.at[pl.ds(sid * MP_ROWS_PER_TILE, MP_ROWS_PER_TILE)],
                    agg_sh.at[pl.ds(sid * MP_ROWS_PER_TILE, MP_ROWS_PER_TILE)])
    pltpu.sync_copy(src_hbm.at[pl.ds(wid * MP_CHUNKS, MP_CHUNKS)], src_idx)
    pltpu.sync_copy(dst_hbm.at[pl.ds(wid * MP_CHUNKS, MP_CHUNKS)], dst_idx)
    plsc.subcore_barrier()

    def body(j, c):
        pltpu.sync_copy(xc_hbm.at[dst_idx.at[j]], rows)
        pltpu.sync_copy(rows, agg_sh.at[src_idx.at[j]], add=True)
        return c

    lax.fori_loop(0, MP_CHUNKS, body, 0)
    plsc.subcore_barrier()
    base = cid * NNP + sid * MP_ROWS_PER_TILE
    pltpu.sync_copy(agg_sh.at[pl.ds(sid * MP_ROWS_PER_TILE, MP_ROWS_PER_TILE)],
                    out_hbm.at[pl.ds(base, MP_ROWS_PER_TILE)])


_mp_call = functools.partial(
    pl.kernel,
    mesh=plsc.VectorSubcoreMesh(core_axis_name="c", subcore_axis_name="s"),
    out_type=jax.ShapeDtypeStruct((NC * NNP, DF), jnp.float32),
    scratch_types=[
        pltpu.VMEM((MP_CHUNKS, MP_K), jnp.int32),
        pltpu.VMEM((MP_CHUNKS, MP_K), jnp.int32),
        pltpu.VMEM((MP_K, DF), jnp.float32),
        pltpu.VMEM_SHARED((NNP, DF), jnp.float32),
    ],
)(_mp_body)


def _node_body(p_ref, w_ref, h_ref):
    agg = p_ref[0] + p_ref[1]
    h_ref[...] = jnp.maximum(
        jnp.dot(agg, w_ref[...], preferred_element_type=jnp.float32), 0.0)


_node = pl.pallas_call(
    _node_body,
    grid=(10,),
    in_specs=[
        pl.BlockSpec((NC, NNP // 10, DF), lambda i: (0, i, 0)),
        pl.BlockSpec((DF, DF), lambda i: (0, 0)),
    ],
    out_specs=pl.BlockSpec((NNP // 10, DF), lambda i: (i, 0)),
    out_shape=jax.ShapeDtypeStruct((NNP, DF), jnp.float32),
)


def _ln_body(h_hbm, x_hbm, u_hbm, v_hbm, ohu, ohv, oxu, oxv,
             u_idx, v_idx, bhu, bhv, bxu, bxv):
    cid = lax.axis_index("c")
    sid = lax.axis_index("s")
    wid = sid * NC + cid
    pltpu.sync_copy(u_hbm.at[pl.ds(wid * LN_CHUNKS, LN_CHUNKS)], u_idx)
    pltpu.sync_copy(v_hbm.at[pl.ds(wid * LN_CHUNKS, LN_CHUNKS)], v_idx)

    def body(j, c):
        base = wid * LN_PER_W + j * LN_K
        pltpu.sync_copy(h_hbm.at[u_idx.at[j]], bhu)
        pltpu.sync_copy(bhu, ohu.at[pl.ds(base, LN_K)])
        pltpu.sync_copy(h_hbm.at[v_idx.at[j]], bhv)
        pltpu.sync_copy(bhv, ohv.at[pl.ds(base, LN_K)])
        pltpu.sync_copy(x_hbm.at[u_idx.at[j]], bxu)
        pltpu.sync_copy(bxu, oxu.at[pl.ds(base, LN_K)])
        pltpu.sync_copy(x_hbm.at[v_idx.at[j]], bxv)
        pltpu.sync_copy(bxv, oxv.at[pl.ds(base, LN_K)])
        return c

    lax.fori_loop(0, LN_CHUNKS, body, 0)


_gathered_sds = jax.ShapeDtypeStruct((NLINK, DF), jnp.float32)
_ln_call = functools.partial(
    pl.kernel,
    mesh=plsc.VectorSubcoreMesh(core_axis_name="c", subcore_axis_name="s"),
    out_type=(_gathered_sds, _gathered_sds, _gathered_sds, _gathered_sds),
    scratch_types=[
        pltpu.VMEM((LN_CHUNKS, LN_K), jnp.int32),
        pltpu.VMEM((LN_CHUNKS, LN_K), jnp.int32),
        pltpu.VMEM((LN_K, DF), jnp.float32),
        pltpu.VMEM((LN_K, DF), jnp.float32),
        pltpu.VMEM((LN_K, DF), jnp.float32),
        pltpu.VMEM((LN_K, DF), jnp.float32),
    ],
)(_ln_body)


def _score_body(hu, hv, xu, xv, ef, wsn, wse, wg, out):
    i = pl.program_id(0)
    huv = hu[...] * hv[...]
    xd = jnp.abs(xu[...] - xv[...])
    experts = (jnp.dot(huv, wsn[...], preferred_element_type=jnp.float32)
               + jnp.dot(ef[...], wse[...], preferred_element_type=jnp.float32))
    gl = jnp.dot(xd, wg[...], preferred_element_type=jnp.float32)
    col = lax.broadcasted_iota(jnp.int32, (SCORE_R, 8), 1)
    gl = jnp.where(col < 4, gl, -1e30)
    m = jnp.max(gl, axis=1, keepdims=True)
    p = jnp.exp(gl - m)
    gate = p / jnp.sum(p, axis=1, keepdims=True)
    logits = jnp.sum(experts * gate, axis=1, keepdims=True)
    r = lax.broadcasted_iota(jnp.int32, (SCORE_R, 1), 0)
    side_row = i * SCORE_R + r - jnp.where(i < SIDE_BLOCKS, 0, NSIDE)
    valid = side_row < NPOS
    tgt = jnp.where(i < SIDE_BLOCKS, 1.0, 0.0)
    bce = (jnp.maximum(logits, 0.0) - logits * tgt
           + jnp.log(1.0 + jnp.exp(-jnp.abs(logits))))
    s = jnp.sum(jnp.where(valid, bce, 0.0))

    @pl.when(i == 0)
    def _():
        out[...] = s.reshape(1, 1)

    @pl.when(i != 0)
    def _():
        out[...] += s.reshape(1, 1)


_score = pl.pallas_call(
    _score_body,
    grid=(SCORE_BLOCKS,),
    in_specs=[
        pl.BlockSpec((SCORE_R, DF), lambda i: (i, 0)),
        pl.BlockSpec((SCORE_R, DF), lambda i: (i, 0)),
        pl.BlockSpec((SCORE_R, DF), lambda i: (i, 0)),
        pl.BlockSpec((SCORE_R, DF), lambda i: (i, 0)),
        pl.BlockSpec((SCORE_R, DE), lambda i: (i, 0)),
        pl.BlockSpec((DF, 8), lambda i: (0, 0)),
        pl.BlockSpec((DE, 8), lambda i: (0, 0)),
        pl.BlockSpec((DF, 8), lambda i: (0, 0)),
    ],
    out_specs=pl.BlockSpec((1, 1), lambda i: (0, 0)),
    out_shape=jax.ShapeDtypeStruct((1, 1), jnp.float32),
)


def kernel(x, mp_link, pos_link, neg_link, pos_feats, neg_feats,
           W1, W_score, W_gate):
    f32 = jnp.float32
    msrc = mp_link[:, 0].reshape(NEDGE // MP_K, MP_K)
    mdst = mp_link[:, 1].reshape(NEDGE // MP_K, MP_K)
    pad_idx = (jnp.arange(NPAD, dtype=jnp.int32) * 13) % NN
    u_all = jnp.concatenate(
        [pos_link[:, 0], pad_idx, neg_link[:, 0], pad_idx]).reshape(
            NLINK // LN_K, LN_K)
    v_all = jnp.concatenate(
        [pos_link[:, 1], pad_idx, neg_link[:, 1], pad_idx]).reshape(
            NLINK // LN_K, LN_K)
    zpad = jnp.zeros((NPAD, DE), f32)
    ef = jnp.concatenate([pos_feats, zpad, neg_feats, zpad], axis=0)
    wsn = jnp.pad(W_score[:DF], ((0, 0), (0, 4)))
    wse = jnp.pad(W_score[DF:], ((0, 0), (0, 4)))
    wg = jnp.pad(W_gate, ((0, 0), (0, 4)))
    ztbl = jnp.zeros((NNP, DF), f32)

    xc = _prep(x)
    partials = _mp_call(xc, msrc, mdst, ztbl)
    h = _node(partials.reshape(NC, NNP, DF), W1)
    hu, hv, xu, xv = _ln_call(h, x, u_all, v_all)
    s = _score(hu, hv, xu, xv, ef, wsn, wse, wg)
    return s[0, 0] * (1.0 / (NPOS + NNEG))


# speedup vs baseline: 2.5495x; 1.0002x over previous
import functools

import jax
import jax.numpy as jnp
from jax import lax
from jax.experimental import pallas as pl
from jax.experimental.pallas import tpu as pltpu
from jax.experimental.pallas import tpu_sc as plsc

NN = 10000
DF = 128
DE = 16
NEDGE = 320000
NPOS = 100000
NNEG = 100000
NSIDE = 102400
NLINK = 2 * NSIDE
NPAD = NSIDE - NPOS

NC = 2
NS = 16
NW = NC * NS

MP_K = 125
MP_CHUNKS = NEDGE // (NW * MP_K)
NNP = 10240
MP_ROWS_PER_TILE = NNP // NS

LN_K = 80
LN_CHUNKS = NLINK // (NW * LN_K)
LN_PER_W = NLINK // NW

SCORE_R = 512
SCORE_BLOCKS = NLINK // SCORE_R
SIDE_BLOCKS = NSIDE // SCORE_R


def _prep_body(x_ref, xc_ref):
    xv = x_ref[...]
    xc_ref[...] = xv - jnp.mean(xv, axis=0, keepdims=True)


_prep = pl.pallas_call(
    _prep_body,
    out_shape=jax.ShapeDtypeStruct((NN, DF), jnp.float32),
)


def _mp_body(xc_hbm, src_hbm, dst_hbm, zero_hbm, out_hbm,
             src_idx, dst_idx, rows, agg_sh):
    cid = lax.axis_index("c")
    sid = lax.axis_index("s")
    wid = sid * NC + cid
    pltpu.sync_copy(zero_hbm.at[pl.ds(sid * MP_ROWS_PER_TILE, MP_ROWS_PER_TILE)],
                    agg_sh.at[pl.ds(sid * MP_ROWS_PER_TILE, MP_ROWS_PER_TILE)])
    pltpu.sync_copy(src_hbm.at[pl.ds(wid * MP_CHUNKS, MP_CHUNKS)], src_idx)
    pltpu.sync_copy(dst_hbm.at[pl.ds(wid * MP_CHUNKS, MP_CHUNKS)], dst_idx)
    plsc.subcore_barrier()

    def body(j, c):
        pltpu.sync_copy(xc_hbm.at[dst_idx.at[j]], rows)
        pltpu.sync_copy(rows, agg_sh.at[src_idx.at[j]], add=True)
        return c

    lax.fori_loop(0, MP_CHUNKS, body, 0)
    plsc.subcore_barrier()
    base = cid * NNP + sid * MP_ROWS_PER_TILE
    pltpu.sync_copy(agg_sh.at[pl.ds(sid * MP_ROWS_PER_TILE, MP_ROWS_PER_TILE)],
                    out_hbm.at[pl.ds(base, MP_ROWS_PER_TILE)])


_mp_call = functools.partial(
    pl.kernel,
    mesh=plsc.VectorSubcoreMesh(core_axis_name="c", subcore_axis_name="s"),
    out_type=jax.ShapeDtypeStruct((NC * NNP, DF), jnp.float32),
    scratch_types=[
        pltpu.VMEM((MP_CHUNKS, MP_K), jnp.int32),
        pltpu.VMEM((MP_CHUNKS, MP_K), jnp.int32),
        pltpu.VMEM((MP_K, DF), jnp.float32),
        pltpu.VMEM_SHARED((NNP, DF), jnp.float32),
    ],
)(_mp_body)


def _node_body(p_ref, w_ref, h_ref):
    agg = p_ref[0] + p_ref[1]
    h_ref[...] = jnp.maximum(
        jnp.dot(agg, w_ref[...], preferred_element_type=jnp.float32), 0.0)


_node = pl.pallas_call(
    _node_body,
    grid=(10,),
    in_specs=[
        pl.BlockSpec((NC, NNP // 10, DF), lambda i: (0, i, 0)),
        pl.BlockSpec((DF, DF), lambda i: (0, 0)),
    ],
    out_specs=pl.BlockSpec((NNP // 10, DF), lambda i: (i, 0)),
    out_shape=jax.ShapeDtypeStruct((NNP, DF), jnp.float32),
)


def _ln_body(h_hbm, x_hbm, u_hbm, v_hbm, ohu, ohv, oxu, oxv,
             u_idx, v_idx, bhu, bhv, bxu, bxv):
    cid = lax.axis_index("c")
    sid = lax.axis_index("s")
    wid = sid * NC + cid
    pltpu.sync_copy(u_hbm.at[pl.ds(wid * LN_CHUNKS, LN_CHUNKS)], u_idx)
    pltpu.sync_copy(v_hbm.at[pl.ds(wid * LN_CHUNKS, LN_CHUNKS)], v_idx)

    def body(j, c):
        base = wid * LN_PER_W + j * LN_K
        pltpu.sync_copy(h_hbm.at[u_idx.at[j]], bhu)
        pltpu.sync_copy(bhu, ohu.at[pl.ds(base, LN_K)])
        pltpu.sync_copy(h_hbm.at[v_idx.at[j]], bhv)
        pltpu.sync_copy(bhv, ohv.at[pl.ds(base, LN_K)])
        pltpu.sync_copy(x_hbm.at[u_idx.at[j]], bxu)
        pltpu.sync_copy(bxu, oxu.at[pl.ds(base, LN_K)])
        pltpu.sync_copy(x_hbm.at[v_idx.at[j]], bxv)
        pltpu.sync_copy(bxv, oxv.at[pl.ds(base, LN_K)])
        return c

    lax.fori_loop(0, LN_CHUNKS, body, 0)


_gathered_sds = jax.ShapeDtypeStruct((NLINK, DF), jnp.float32)
_ln_call = functools.partial(
    pl.kernel,
    mesh=plsc.VectorSubcoreMesh(core_axis_name="c", subcore_axis_name="s"),
    out_type=(_gathered_sds, _gathered_sds, _gathered_sds, _gathered_sds),
    scratch_types=[
        pltpu.VMEM((LN_CHUNKS, LN_K), jnp.int32),
        pltpu.VMEM((LN_CHUNKS, LN_K), jnp.int32),
        pltpu.VMEM((LN_K, DF), jnp.float32),
        pltpu.VMEM((LN_K, DF), jnp.float32),
        pltpu.VMEM((LN_K, DF), jnp.float32),
        pltpu.VMEM((LN_K, DF), jnp.float32),
    ],
)(_ln_body)


def _score_body(hu, hv, xu, xv, ef, wsn, wse, wg, out):
    i = pl.program_id(0)
    huv = hu[...].astype(jnp.float32) * hv[...].astype(jnp.float32)
    xd = jnp.abs(xu[...].astype(jnp.float32) - xv[...].astype(jnp.float32))
    experts = (jnp.dot(huv, wsn[...], preferred_element_type=jnp.float32)
               + jnp.dot(ef[...], wse[...], preferred_element_type=jnp.float32))
    gl = jnp.dot(xd, wg[...], preferred_element_type=jnp.float32)
    col = lax.broadcasted_iota(jnp.int32, (SCORE_R, 8), 1)
    gl = jnp.where(col < 4, gl, -1e30)
    m = jnp.max(gl, axis=1, keepdims=True)
    p = jnp.exp(gl - m)
    gate = p / jnp.sum(p, axis=1, keepdims=True)
    logits = jnp.sum(experts * gate, axis=1, keepdims=True)
    r = lax.broadcasted_iota(jnp.int32, (SCORE_R, 1), 0)
    side_row = i * SCORE_R + r - jnp.where(i < SIDE_BLOCKS, 0, NSIDE)
    valid = side_row < NPOS
    tgt = jnp.where(i < SIDE_BLOCKS, 1.0, 0.0)
    bce = (jnp.maximum(logits, 0.0) - logits * tgt
           + jnp.log(1.0 + jnp.exp(-jnp.abs(logits))))
    s = jnp.sum(jnp.where(valid, bce, 0.0))

    @pl.when(i == 0)
    def _():
        out[...] = s.reshape(1, 1)

    @pl.when(i != 0)
    def _():
        out[...] += s.reshape(1, 1)


_score = pl.pallas_call(
    _score_body,
    grid=(SCORE_BLOCKS,),
    in_specs=[
        pl.BlockSpec((SCORE_R, DF), lambda i: (i, 0)),
        pl.BlockSpec((SCORE_R, DF), lambda i: (i, 0)),
        pl.BlockSpec((SCORE_R, DF), lambda i: (i, 0)),
        pl.BlockSpec((SCORE_R, DF), lambda i: (i, 0)),
        pl.BlockSpec((SCORE_R, DE), lambda i: (i, 0)),
        pl.BlockSpec((DF, 8), lambda i: (0, 0)),
        pl.BlockSpec((DE, 8), lambda i: (0, 0)),
        pl.BlockSpec((DF, 8), lambda i: (0, 0)),
    ],
    out_specs=pl.BlockSpec((1, 1), lambda i: (0, 0)),
    out_shape=jax.ShapeDtypeStruct((1, 1), jnp.float32),
)


def kernel(x, mp_link, pos_link, neg_link, pos_feats, neg_feats,
           W1, W_score, W_gate):
    f32 = jnp.float32
    msrc = mp_link[:, 0].reshape(NEDGE // MP_K, MP_K)
    mdst = mp_link[:, 1].reshape(NEDGE // MP_K, MP_K)
    pad_idx = (jnp.arange(NPAD, dtype=jnp.int32) * 13) % NN
    u_all = jnp.concatenate(
        [pos_link[:, 0], pad_idx, neg_link[:, 0], pad_idx]).reshape(
            NLINK // LN_K, LN_K)
    v_all = jnp.concatenate(
        [pos_link[:, 1], pad_idx, neg_link[:, 1], pad_idx]).reshape(
            NLINK // LN_K, LN_K)
    zpad = jnp.zeros((NPAD, DE), f32)
    ef = jnp.concatenate([pos_feats, zpad, neg_feats, zpad], axis=0)
    wsn = jnp.pad(W_score[:DF], ((0, 0), (0, 4)))
    wse = jnp.pad(W_score[DF:], ((0, 0), (0, 4)))
    wg = jnp.pad(W_gate, ((0, 0), (0, 4)))
    ztbl = jnp.zeros((NNP, DF), f32)

    xc = _prep(x)
    partials = _mp_call(xc, msrc, mdst, ztbl)
    h = _node(partials.reshape(NC, NNP, DF), W1)
    hu, hv, xu, xv = _ln_call(h, x, u_all, v_all)
    s = _score(hu, hv, xu, xv, ef, wsn, wse, wg)
    return s[0, 0] * (1.0 / (NPOS + NNEG))


# speedup vs baseline: 3.0449x; 1.1943x over previous
import functools

import jax
import jax.numpy as jnp
from jax import lax
from jax.experimental import pallas as pl
from jax.experimental.pallas import tpu as pltpu
from jax.experimental.pallas import tpu_sc as plsc

NN = 10000
DF = 128
DE = 16
NEDGE = 320000
NPOS = 100000
NNEG = 100000
NSIDE = 102400
NLINK = 2 * NSIDE
NPAD = NSIDE - NPOS

NC = 2
NS = 16
NW = NC * NS

MP_K = 125
MP_CHUNKS = NEDGE // (NW * MP_K)
NNP = 10240
MP_ROWS_PER_TILE = NNP // NS

LN_K = 80
LN_CHUNKS = NLINK // (NW * LN_K)
LN_PER_W = NLINK // NW

SCORE_R = 512
SCORE_BLOCKS = NLINK // SCORE_R
SIDE_BLOCKS = NSIDE // SCORE_R


def _prep_body(x_ref, xc_ref):
    xv = x_ref[...]
    xc_ref[...] = xv - jnp.mean(xv, axis=0, keepdims=True)


_prep = pl.pallas_call(
    _prep_body,
    out_shape=jax.ShapeDtypeStruct((NN, DF), jnp.float32),
)


def _mp_body(xc_hbm, src_hbm, dst_hbm, zero_hbm, out_hbm,
             src_idx, dst_idx, rows, agg_sh):
    cid = lax.axis_index("c")
    sid = lax.axis_index("s")
    wid = sid * NC + cid
    pltpu.sync_copy(zero_hbm.at[pl.ds(sid * MP_ROWS_PER_TILE, MP_ROWS_PER_TILE)],
                    agg_sh.at[pl.ds(sid * MP_ROWS_PER_TILE, MP_ROWS_PER_TILE)])
    pltpu.sync_copy(src_hbm.at[pl.ds(wid * MP_CHUNKS, MP_CHUNKS)], src_idx)
    pltpu.sync_copy(dst_hbm.at[pl.ds(wid * MP_CHUNKS, MP_CHUNKS)], dst_idx)
    plsc.subcore_barrier()

    def body(j, c):
        pltpu.sync_copy(xc_hbm.at[dst_idx.at[j]], rows)
        pltpu.sync_copy(rows, agg_sh.at[src_idx.at[j]], add=True)
        return c

    lax.fori_loop(0, MP_CHUNKS, body, 0)
    plsc.subcore_barrier()
    base = cid * NNP + sid * MP_ROWS_PER_TILE
    pltpu.sync_copy(agg_sh.at[pl.ds(sid * MP_ROWS_PER_TILE, MP_ROWS_PER_TILE)],
                    out_hbm.at[pl.ds(base, MP_ROWS_PER_TILE)])


_mp_call = functools.partial(
    pl.kernel,
    mesh=plsc.VectorSubcoreMesh(core_axis_name="c", subcore_axis_name="s"),
    out_type=jax.ShapeDtypeStruct((NC * NNP, DF), jnp.float32),
    scratch_types=[
        pltpu.VMEM((MP_CHUNKS, MP_K), jnp.int32),
        pltpu.VMEM((MP_CHUNKS, MP_K), jnp.int32),
        pltpu.VMEM((MP_K, DF), jnp.float32),
        pltpu.VMEM_SHARED((NNP, DF), jnp.float32),
    ],
)(_mp_body)


def _node_body(p_ref, w_ref, h_ref):
    agg = p_ref[0] + p_ref[1]
    h_ref[...] = jnp.maximum(
        jnp.dot(agg, w_ref[...], preferred_element_type=jnp.float32), 0.0)


_node = pl.pallas_call(
    _node_body,
    grid=(10,),
    in_specs=[
        pl.BlockSpec((NC, NNP // 10, DF), lambda i: (0, i, 0)),
        pl.BlockSpec((DF, DF), lambda i: (0, 0)),
    ],
    out_specs=pl.BlockSpec((NNP // 10, DF), lambda i: (i, 0)),
    out_shape=jax.ShapeDtypeStruct((NNP, DF), jnp.float32),
)


def _ln_body(h_hbm, x_hbm, u_hbm, v_hbm, ohu, ohv, oxu, oxv,
             u_idx, v_idx,
             a_hu, a_hv, a_xu, a_xv, b_hu, b_hv, b_xu, b_xv,
             sga, sgb, swa, swb):
    cid = lax.axis_index("c")
    sid = lax.axis_index("s")
    wid = sid * NC + cid
    pltpu.sync_copy(u_hbm.at[pl.ds(wid * LN_CHUNKS, LN_CHUNKS)], u_idx)
    pltpu.sync_copy(v_hbm.at[pl.ds(wid * LN_CHUNKS, LN_CHUNKS)], v_idx)

    bufs_a = (a_hu, a_hv, a_xu, a_xv)
    bufs_b = (b_hu, b_hv, b_xu, b_xv)
    outs = (ohu, ohv, oxu, oxv)

    def fire_gathers(j, bufs, sem):
        pltpu.async_copy(h_hbm.at[u_idx.at[j]], bufs[0], sem)
        pltpu.async_copy(h_hbm.at[v_idx.at[j]], bufs[1], sem)
        pltpu.async_copy(x_hbm.at[u_idx.at[j]], bufs[2], sem)
        pltpu.async_copy(x_hbm.at[v_idx.at[j]], bufs[3], sem)

    def drain_gathers(j, bufs, sem):
        pltpu.make_async_copy(h_hbm.at[u_idx.at[j]], bufs[0], sem).wait()
        pltpu.make_async_copy(h_hbm.at[v_idx.at[j]], bufs[1], sem).wait()
        pltpu.make_async_copy(x_hbm.at[u_idx.at[j]], bufs[2], sem).wait()
        pltpu.make_async_copy(x_hbm.at[v_idx.at[j]], bufs[3], sem).wait()

    def fire_writes(j, bufs, sem):
        base = wid * LN_PER_W + j * LN_K
        for t in range(4):
            pltpu.async_copy(bufs[t], outs[t].at[pl.ds(base, LN_K)], sem)

    def drain_writes(j, bufs, sem):
        base = wid * LN_PER_W + j * LN_K
        for t in range(4):
            pltpu.make_async_copy(
                bufs[t], outs[t].at[pl.ds(base, LN_K)], sem).wait()

    def pair(p, c):
        j0 = 2 * p
        j1 = j0 + 1

        @pl.when(p > 0)
        def _():
            drain_writes(j0 - 2, bufs_a, swa)

        fire_gathers(j0, bufs_a, sga)

        @pl.when(p > 0)
        def _():
            drain_writes(j0 - 1, bufs_b, swb)

        fire_gathers(j1, bufs_b, sgb)
        drain_gathers(j0, bufs_a, sga)
        fire_writes(j0, bufs_a, swa)
        drain_gathers(j1, bufs_b, sgb)
        fire_writes(j1, bufs_b, swb)
        return c

    lax.fori_loop(0, LN_CHUNKS // 2, pair, 0)
    drain_writes(LN_CHUNKS - 2, bufs_a, swa)
    drain_writes(LN_CHUNKS - 1, bufs_b, swb)


_gathered_sds = jax.ShapeDtypeStruct((NLINK, DF), jnp.float32)
_ln_call = functools.partial(
    pl.kernel,
    mesh=plsc.VectorSubcoreMesh(core_axis_name="c", subcore_axis_name="s"),
    out_type=(_gathered_sds, _gathered_sds, _gathered_sds, _gathered_sds),
    scratch_types=(
        [pltpu.VMEM((LN_CHUNKS, LN_K), jnp.int32)] * 2
        + [pltpu.VMEM((LN_K, DF), jnp.float32)] * 8
        + [pltpu.SemaphoreType.DMA] * 4
    ),
)(_ln_body)


def _score_body(hu, hv, xu, xv, ef, wsn, wse, wg, out):
    i = pl.program_id(0)
    huv = hu[...].astype(jnp.float32) * hv[...].astype(jnp.float32)
    xd = jnp.abs(xu[...].astype(jnp.float32) - xv[...].astype(jnp.float32))
    experts = (jnp.dot(huv, wsn[...], preferred_element_type=jnp.float32)
               + jnp.dot(ef[...], wse[...], preferred_element_type=jnp.float32))
    gl = jnp.dot(xd, wg[...], preferred_element_type=jnp.float32)
    col = lax.broadcasted_iota(jnp.int32, (SCORE_R, 8), 1)
    gl = jnp.where(col < 4, gl, -1e30)
    m = jnp.max(gl, axis=1, keepdims=True)
    p = jnp.exp(gl - m)
    gate = p / jnp.sum(p, axis=1, keepdims=True)
    logits = jnp.sum(experts * gate, axis=1, keepdims=True)
    r = lax.broadcasted_iota(jnp.int32, (SCORE_R, 1), 0)
    side_row = i * SCORE_R + r - jnp.where(i < SIDE_BLOCKS, 0, NSIDE)
    valid = side_row < NPOS
    tgt = jnp.where(i < SIDE_BLOCKS, 1.0, 0.0)
    bce = (jnp.maximum(logits, 0.0) - logits * tgt
           + jnp.log(1.0 + jnp.exp(-jnp.abs(logits))))
    s = jnp.sum(jnp.where(valid, bce, 0.0))

    @pl.when(i == 0)
    def _():
        out[...] = s.reshape(1, 1)

    @pl.when(i != 0)
    def _():
        out[...] += s.reshape(1, 1)


_score = pl.pallas_call(
    _score_body,
    grid=(SCORE_BLOCKS,),
    in_specs=[
        pl.BlockSpec((SCORE_R, DF), lambda i: (i, 0)),
        pl.BlockSpec((SCORE_R, DF), lambda i: (i, 0)),
        pl.BlockSpec((SCORE_R, DF), lambda i: (i, 0)),
        pl.BlockSpec((SCORE_R, DF), lambda i: (i, 0)),
        pl.BlockSpec((SCORE_R, DE), lambda i: (i, 0)),
        pl.BlockSpec((DF, 8), lambda i: (0, 0)),
        pl.BlockSpec((DE, 8), lambda i: (0, 0)),
        pl.BlockSpec((DF, 8), lambda i: (0, 0)),
    ],
    out_specs=pl.BlockSpec((1, 1), lambda i: (0, 0)),
    out_shape=jax.ShapeDtypeStruct((1, 1), jnp.float32),
)


def kernel(x, mp_link, pos_link, neg_link, pos_feats, neg_feats,
           W1, W_score, W_gate):
    f32 = jnp.float32
    msrc = mp_link[:, 0].reshape(NEDGE // MP_K, MP_K)
    mdst = mp_link[:, 1].reshape(NEDGE // MP_K, MP_K)
    pad_idx = (jnp.arange(NPAD, dtype=jnp.int32) * 13) % NN
    u_all = jnp.concatenate(
        [pos_link[:, 0], pad_idx, neg_link[:, 0], pad_idx]).reshape(
            NLINK // LN_K, LN_K)
    v_all = jnp.concatenate(
        [pos_link[:, 1], pad_idx, neg_link[:, 1], pad_idx]).reshape(
            NLINK // LN_K, LN_K)
    zpad = jnp.zeros((NPAD, DE), f32)
    ef = jnp.concatenate([pos_feats, zpad, neg_feats, zpad], axis=0)
    wsn = jnp.pad(W_score[:DF], ((0, 0), (0, 4)))
    wse = jnp.pad(W_score[DF:], ((0, 0), (0, 4)))
    wg = jnp.pad(W_gate, ((0, 0), (0, 4)))
    ztbl = jnp.zeros((NNP, DF), f32)

    xc = _prep(x)
    partials = _mp_call(xc, msrc, mdst, ztbl)
    h = _node(partials.reshape(NC, NNP, DF), W1)
    hu, hv, xu, xv = _ln_call(h, x, u_all, v_all)
    s = _score(hu, hv, xu, xv, ef, wsn, wse, wg)
    return s[0, 0] * (1.0 / (NPOS + NNEG))


# speedup vs baseline: 3.1380x; 1.0306x over previous
import functools

import jax
import jax.numpy as jnp
from jax import lax
from jax.experimental import pallas as pl
from jax.experimental.pallas import tpu as pltpu
from jax.experimental.pallas import tpu_sc as plsc

NN = 10000
DF = 128
DE = 16
NEDGE = 320000
NPOS = 100000
NNEG = 100000
NSIDE = 102400
NLINK = 2 * NSIDE
NPAD = NSIDE - NPOS

NC = 2
NS = 16
NW = NC * NS

MP_K = 125
MP_CHUNKS = NEDGE // (NW * MP_K)
MP_GCH = 40
NNP = 10240
MP_ROWS_PER_TILE = NNP // NS

LN_K = 80
LN_CHUNKS = NLINK // (NW * LN_K)
LN_PER_W = NLINK // NW

SCORE_R = 512
SCORE_BLOCKS = NLINK // SCORE_R
SIDE_BLOCKS = NSIDE // SCORE_R


def _prep_body(x_ref, xc_ref):
    xv = x_ref[...]
    xc_ref[...] = xv - jnp.mean(xv, axis=0, keepdims=True)


_prep = pl.pallas_call(
    _prep_body,
    out_shape=jax.ShapeDtypeStruct((NN, DF), jnp.float32),
)


def _mp_body(xc_hbm, src_hbm, dst_hbm, zero_hbm, out_hbm,
             src_idx, dst_idx, rows_a, rows_b, agg_sh, ga, gb, sa, sb):
    cid = lax.axis_index("c")
    sid = lax.axis_index("s")
    wid = sid * NC + cid
    pltpu.sync_copy(zero_hbm.at[pl.ds(sid * MP_ROWS_PER_TILE, MP_ROWS_PER_TILE)],
                    agg_sh.at[pl.ds(sid * MP_ROWS_PER_TILE, MP_ROWS_PER_TILE)])
    plsc.subcore_barrier()

    def drain_scatter(j, rows, sem):
        pltpu.make_async_copy(rows, agg_sh.at[src_idx.at[j]], sem).wait()

    def group(g, c):
        base = wid * MP_CHUNKS + g * MP_GCH
        pltpu.sync_copy(src_hbm.at[pl.ds(base, MP_GCH)], src_idx)
        pltpu.sync_copy(dst_hbm.at[pl.ds(base, MP_GCH)], dst_idx)

        def pair(p, c2):
            j0 = 2 * p
            j1 = j0 + 1

            @pl.when(p > 0)
            def _():
                drain_scatter(j0 - 2, rows_a, sa)

            pltpu.async_copy(xc_hbm.at[dst_idx.at[j0]], rows_a, ga)

            @pl.when(p > 0)
            def _():
                drain_scatter(j0 - 1, rows_b, sb)

            pltpu.async_copy(xc_hbm.at[dst_idx.at[j1]], rows_b, gb)
            pltpu.make_async_copy(xc_hbm.at[dst_idx.at[j0]], rows_a, ga).wait()
            pltpu.async_copy(rows_a, agg_sh.at[src_idx.at[j0]], sa, add=True)
            pltpu.make_async_copy(xc_hbm.at[dst_idx.at[j1]], rows_b, gb).wait()
            pltpu.async_copy(rows_b, agg_sh.at[src_idx.at[j1]], sb, add=True)
            return c2

        lax.fori_loop(0, MP_GCH // 2, pair, 0)
        drain_scatter(MP_GCH - 2, rows_a, sa)
        drain_scatter(MP_GCH - 1, rows_b, sb)
        return c

    lax.fori_loop(0, MP_CHUNKS // MP_GCH, group, 0)
    plsc.subcore_barrier()
    base = cid * NNP + sid * MP_ROWS_PER_TILE
    pltpu.sync_copy(agg_sh.at[pl.ds(sid * MP_ROWS_PER_TILE, MP_ROWS_PER_TILE)],
                    out_hbm.at[pl.ds(base, MP_ROWS_PER_TILE)])


_mp_call = functools.partial(
    pl.kernel,
    mesh=plsc.VectorSubcoreMesh(core_axis_name="c", subcore_axis_name="s"),
    out_type=jax.ShapeDtypeStruct((NC * NNP, DF), jnp.float32),
    scratch_types=[
        pltpu.VMEM((MP_GCH, MP_K), jnp.int32),
        pltpu.VMEM((MP_GCH, MP_K), jnp.int32),
        pltpu.VMEM((MP_K, DF), jnp.float32),
        pltpu.VMEM((MP_K, DF), jnp.float32),
        pltpu.VMEM_SHARED((NNP, DF), jnp.float32),
        pltpu.SemaphoreType.DMA,
        pltpu.SemaphoreType.DMA,
        pltpu.SemaphoreType.DMA,
        pltpu.SemaphoreType.DMA,
    ],
)(_mp_body)


def _node_body(p_ref, w_ref, h_ref):
    agg = p_ref[0] + p_ref[1]
    h_ref[...] = jnp.maximum(
        jnp.dot(agg, w_ref[...], preferred_element_type=jnp.float32), 0.0)


_node = pl.pallas_call(
    _node_body,
    grid=(10,),
    in_specs=[
        pl.BlockSpec((NC, NNP // 10, DF), lambda i: (0, i, 0)),
        pl.BlockSpec((DF, DF), lambda i: (0, 0)),
    ],
    out_specs=pl.BlockSpec((NNP // 10, DF), lambda i: (i, 0)),
    out_shape=jax.ShapeDtypeStruct((NNP, DF), jnp.float32),
)


def _ln_body(h_hbm, x_hbm, u_hbm, v_hbm, ohu, ohv, oxu, oxv,
             u_idx, v_idx,
             a_hu, a_hv, a_xu, a_xv, b_hu, b_hv, b_xu, b_xv,
             sga, sgb, swa, swb):
    cid = lax.axis_index("c")
    sid = lax.axis_index("s")
    wid = sid * NC + cid
    pltpu.sync_copy(u_hbm.at[pl.ds(wid * LN_CHUNKS, LN_CHUNKS)], u_idx)
    pltpu.sync_copy(v_hbm.at[pl.ds(wid * LN_CHUNKS, LN_CHUNKS)], v_idx)

    bufs_a = (a_hu, a_hv, a_xu, a_xv)
    bufs_b = (b_hu, b_hv, b_xu, b_xv)
    outs = (ohu, ohv, oxu, oxv)

    def fire_gathers(j, bufs, sem):
        pltpu.async_copy(h_hbm.at[u_idx.at[j]], bufs[0], sem)
        pltpu.async_copy(h_hbm.at[v_idx.at[j]], bufs[1], sem)
        pltpu.async_copy(x_hbm.at[u_idx.at[j]], bufs[2], sem)
        pltpu.async_copy(x_hbm.at[v_idx.at[j]], bufs[3], sem)

    def drain_gathers(j, bufs, sem):
        pltpu.make_async_copy(h_hbm.at[u_idx.at[j]], bufs[0], sem).wait()
        pltpu.make_async_copy(h_hbm.at[v_idx.at[j]], bufs[1], sem).wait()
        pltpu.make_async_copy(x_hbm.at[u_idx.at[j]], bufs[2], sem).wait()
        pltpu.make_async_copy(x_hbm.at[v_idx.at[j]], bufs[3], sem).wait()

    def fire_writes(j, bufs, sem):
        base = wid * LN_PER_W + j * LN_K
        for t in range(4):
            pltpu.async_copy(bufs[t], outs[t].at[pl.ds(base, LN_K)], sem)

    def drain_writes(j, bufs, sem):
        base = wid * LN_PER_W + j * LN_K
        for t in range(4):
            pltpu.make_async_copy(
                bufs[t], outs[t].at[pl.ds(base, LN_K)], sem).wait()

    def pair(p, c):
        j0 = 2 * p
        j1 = j0 + 1

        @pl.when(p > 0)
        def _():
            drain_writes(j0 - 2, bufs_a, swa)

        fire_gathers(j0, bufs_a, sga)

        @pl.when(p > 0)
        def _():
            drain_writes(j0 - 1, bufs_b, swb)

        fire_gathers(j1, bufs_b, sgb)
        drain_gathers(j0, bufs_a, sga)
        fire_writes(j0, bufs_a, swa)
        drain_gathers(j1, bufs_b, sgb)
        fire_writes(j1, bufs_b, swb)
        return c

    lax.fori_loop(0, LN_CHUNKS // 2, pair, 0)
    drain_writes(LN_CHUNKS - 2, bufs_a, swa)
    drain_writes(LN_CHUNKS - 1, bufs_b, swb)


_gathered_sds = jax.ShapeDtypeStruct((NLINK, DF), jnp.float32)
_ln_call = functools.partial(
    pl.kernel,
    mesh=plsc.VectorSubcoreMesh(core_axis_name="c", subcore_axis_name="s"),
    out_type=(_gathered_sds, _gathered_sds, _gathered_sds, _gathered_sds),
    scratch_types=(
        [pltpu.VMEM((LN_CHUNKS, LN_K), jnp.int32)] * 2
        + [pltpu.VMEM((LN_K, DF), jnp.float32)] * 8
        + [pltpu.SemaphoreType.DMA] * 4
    ),
)(_ln_body)


def _score_body(hu, hv, xu, xv, ef, wsn, wse, wg, out):
    i = pl.program_id(0)
    huv = hu[...].astype(jnp.float32) * hv[...].astype(jnp.float32)
    xd = jnp.abs(xu[...].astype(jnp.float32) - xv[...].astype(jnp.float32))
    experts = (jnp.dot(huv, wsn[...], preferred_element_type=jnp.float32)
               + jnp.dot(ef[...], wse[...], preferred_element_type=jnp.float32))
    gl = jnp.dot(xd, wg[...], preferred_element_type=jnp.float32)
    col = lax.broadcasted_iota(jnp.int32, (SCORE_R, 8), 1)
    gl = jnp.where(col < 4, gl, -1e30)
    m = jnp.max(gl, axis=1, keepdims=True)
    p = jnp.exp(gl - m)
    gate = p / jnp.sum(p, axis=1, keepdims=True)
    logits = jnp.sum(experts * gate, axis=1, keepdims=True)
    r = lax.broadcasted_iota(jnp.int32, (SCORE_R, 1), 0)
    side_row = i * SCORE_R + r - jnp.where(i < SIDE_BLOCKS, 0, NSIDE)
    valid = side_row < NPOS
    tgt = jnp.where(i < SIDE_BLOCKS, 1.0, 0.0)
    bce = (jnp.maximum(logits, 0.0) - logits * tgt
           + jnp.log(1.0 + jnp.exp(-jnp.abs(logits))))
    s = jnp.sum(jnp.where(valid, bce, 0.0))

    @pl.when(i == 0)
    def _():
        out[...] = s.reshape(1, 1)

    @pl.when(i != 0)
    def _():
        out[...] += s.reshape(1, 1)


_score = pl.pallas_call(
    _score_body,
    grid=(SCORE_BLOCKS,),
    in_specs=[
        pl.BlockSpec((SCORE_R, DF), lambda i: (i, 0)),
        pl.BlockSpec((SCORE_R, DF), lambda i: (i, 0)),
        pl.BlockSpec((SCORE_R, DF), lambda i: (i, 0)),
        pl.BlockSpec((SCORE_R, DF), lambda i: (i, 0)),
        pl.BlockSpec((SCORE_R, DE), lambda i: (i, 0)),
        pl.BlockSpec((DF, 8), lambda i: (0, 0)),
        pl.BlockSpec((DE, 8), lambda i: (0, 0)),
        pl.BlockSpec((DF, 8), lambda i: (0, 0)),
    ],
    out_specs=pl.BlockSpec((1, 1), lambda i: (0, 0)),
    out_shape=jax.ShapeDtypeStruct((1, 1), jnp.float32),
)


def kernel(x, mp_link, pos_link, neg_link, pos_feats, neg_feats,
           W1, W_score, W_gate):
    f32 = jnp.float32
    msrc = mp_link[:, 0].reshape(NEDGE // MP_K, MP_K)
    mdst = mp_link[:, 1].reshape(NEDGE // MP_K, MP_K)
    pad_idx = (jnp.arange(NPAD, dtype=jnp.int32) * 13) % NN
    u_all = jnp.concatenate(
        [pos_link[:, 0], pad_idx, neg_link[:, 0], pad_idx]).reshape(
            NLINK // LN_K, LN_K)
    v_all = jnp.concatenate(
        [pos_link[:, 1], pad_idx, neg_link[:, 1], pad_idx]).reshape(
            NLINK // LN_K, LN_K)
    zpad = jnp.zeros((NPAD, DE), f32)
    ef = jnp.concatenate([pos_feats, zpad, neg_feats, zpad], axis=0)
    wsn = jnp.pad(W_score[:DF], ((0, 0), (0, 4)))
    wse = jnp.pad(W_score[DF:], ((0, 0), (0, 4)))
    wg = jnp.pad(W_gate, ((0, 0), (0, 4)))
    ztbl = jnp.zeros((NNP, DF), f32)

    xc = _prep(x)
    partials = _mp_call(xc, msrc, mdst, ztbl)
    h = _node(partials.reshape(NC, NNP, DF), W1)
    hu, hv, xu, xv = _ln_call(h, x, u_all, v_all)
    s = _score(hu, hv, xu, xv, ef, wsn, wse, wg)
    return s[0, 0] * (1.0 / (NPOS + NNEG))


# speedup vs baseline: 3.8350x; 1.2221x over previous
import functools

import jax
import jax.numpy as jnp
from jax import lax
from jax.experimental import pallas as pl
from jax.experimental.pallas import tpu as pltpu
from jax.experimental.pallas import tpu_sc as plsc

NN = 10000
DF = 128
DE = 16
NEDGE = 320000
NPOS = 100000
NNEG = 100000
NSIDE = 102400
NLINK = 2 * NSIDE
NPAD = NSIDE - NPOS

NC = 2
NS = 16
NW = NC * NS

MP_K = 125
MP_CHUNKS = NEDGE // (NW * MP_K)
MP_GCH = 40
NNP = 10240
MP_ROWS_PER_TILE = NNP // NS

LN_K = 80
LN_CHUNKS = NLINK // (NW * LN_K)
LN_PER_W = NLINK // NW

SCORE_R = 512
SCORE_BLOCKS = NLINK // SCORE_R
SIDE_BLOCKS = NSIDE // SCORE_R


DP = DF // 2


def _rne(u):
    return u + jnp.uint32(0x7FFF) + ((u >> 16) & jnp.uint32(1))


def _pack(t):
    u = lax.bitcast_convert_type(t, jnp.uint32)
    w = (_rne(u[:, :DP]) & jnp.uint32(0xFFFF0000)) | (_rne(u[:, DP:]) >> 16)
    return lax.bitcast_convert_type(w, jnp.int32)


def _unpack(p):
    u = lax.bitcast_convert_type(p, jnp.uint32)
    hi = lax.bitcast_convert_type(u & jnp.uint32(0xFFFF0000), jnp.float32)
    lo = lax.bitcast_convert_type(u << 16, jnp.float32)
    return jnp.concatenate([hi, lo], axis=1)


def _prep_body(x_ref, xc_ref):
    xv = x_ref[...]
    xc_ref[...] = xv - jnp.mean(xv, axis=0, keepdims=True)


_prep = pl.pallas_call(
    _prep_body,
    out_shape=jax.ShapeDtypeStruct((NN, DF), jnp.float32),
)


def _mp_body(xc_hbm, src_hbm, dst_hbm, zero_hbm, out_hbm,
             src_idx, dst_idx, rows_a, rows_b, agg_sh, ga, gb, sa, sb):
    cid = lax.axis_index("c")
    sid = lax.axis_index("s")
    wid = sid * NC + cid
    pltpu.sync_copy(zero_hbm.at[pl.ds(sid * MP_ROWS_PER_TILE, MP_ROWS_PER_TILE)],
                    agg_sh.at[pl.ds(sid * MP_ROWS_PER_TILE, MP_ROWS_PER_TILE)])
    plsc.subcore_barrier()

    def drain_scatter(j, rows, sem):
        pltpu.make_async_copy(rows, agg_sh.at[src_idx.at[j]], sem).wait()

    def group(g, c):
        base = wid * MP_CHUNKS + g * MP_GCH
        pltpu.sync_copy(src_hbm.at[pl.ds(base, MP_GCH)], src_idx)
        pltpu.sync_copy(dst_hbm.at[pl.ds(base, MP_GCH)], dst_idx)

        def pair(p, c2):
            j0 = 2 * p
            j1 = j0 + 1

            @pl.when(p > 0)
            def _():
                drain_scatter(j0 - 2, rows_a, sa)

            pltpu.async_copy(xc_hbm.at[dst_idx.at[j0]], rows_a, ga)

            @pl.when(p > 0)
            def _():
                drain_scatter(j0 - 1, rows_b, sb)

            pltpu.async_copy(xc_hbm.at[dst_idx.at[j1]], rows_b, gb)
            pltpu.make_async_copy(xc_hbm.at[dst_idx.at[j0]], rows_a, ga).wait()
            pltpu.async_copy(rows_a, agg_sh.at[src_idx.at[j0]], sa, add=True)
            pltpu.make_async_copy(xc_hbm.at[dst_idx.at[j1]], rows_b, gb).wait()
            pltpu.async_copy(rows_b, agg_sh.at[src_idx.at[j1]], sb, add=True)
            return c2

        lax.fori_loop(0, MP_GCH // 2, pair, 0)
        drain_scatter(MP_GCH - 2, rows_a, sa)
        drain_scatter(MP_GCH - 1, rows_b, sb)
        return c

    lax.fori_loop(0, MP_CHUNKS // MP_GCH, group, 0)
    plsc.subcore_barrier()
    base = cid * NNP + sid * MP_ROWS_PER_TILE
    pltpu.sync_copy(agg_sh.at[pl.ds(sid * MP_ROWS_PER_TILE, MP_ROWS_PER_TILE)],
                    out_hbm.at[pl.ds(base, MP_ROWS_PER_TILE)])


_mp_call = functools.partial(
    pl.kernel,
    mesh=plsc.VectorSubcoreMesh(core_axis_name="c", subcore_axis_name="s"),
    out_type=jax.ShapeDtypeStruct((NC * NNP, DF), jnp.float32),
    scratch_types=[
        pltpu.VMEM((MP_GCH, MP_K), jnp.int32),
        pltpu.VMEM((MP_GCH, MP_K), jnp.int32),
        pltpu.VMEM((MP_K, DF), jnp.float32),
        pltpu.VMEM((MP_K, DF), jnp.float32),
        pltpu.VMEM_SHARED((NNP, DF), jnp.float32),
        pltpu.SemaphoreType.DMA,
        pltpu.SemaphoreType.DMA,
        pltpu.SemaphoreType.DMA,
        pltpu.SemaphoreType.DMA,
    ],
)(_mp_body)


def _node_body(p_ref, w_ref, xp_ref, hx_ref):
    agg = p_ref[0] + p_ref[1]
    h = jnp.maximum(
        jnp.dot(agg, w_ref[...], preferred_element_type=jnp.float32), 0.0)
    hx_ref[...] = jnp.concatenate([_pack(h), _pack(xp_ref[...])], axis=1)


_node = pl.pallas_call(
    _node_body,
    grid=(10,),
    in_specs=[
        pl.BlockSpec((NC, NNP // 10, DF), lambda i: (0, i, 0)),
        pl.BlockSpec((DF, DF), lambda i: (0, 0)),
        pl.BlockSpec((NNP // 10, DF), lambda i: (i, 0)),
    ],
    out_specs=pl.BlockSpec((NNP // 10, DF), lambda i: (i, 0)),
    out_shape=jax.ShapeDtypeStruct((NNP, DF), jnp.int32),
)


def _ln_body(hx_hbm, u_hbm, v_hbm, ohxu, ohxv,
             u_idx, v_idx, a_u, a_v, b_u, b_v, sga, sgb, swa, swb):
    cid = lax.axis_index("c")
    sid = lax.axis_index("s")
    wid = sid * NC + cid
    pltpu.sync_copy(u_hbm.at[pl.ds(wid * LN_CHUNKS, LN_CHUNKS)], u_idx)
    pltpu.sync_copy(v_hbm.at[pl.ds(wid * LN_CHUNKS, LN_CHUNKS)], v_idx)

    bufs_a = (a_u, a_v)
    bufs_b = (b_u, b_v)
    outs = (ohxu, ohxv)

    def fire_gathers(j, bufs, sem):
        pltpu.async_copy(hx_hbm.at[u_idx.at[j]], bufs[0], sem)
        pltpu.async_copy(hx_hbm.at[v_idx.at[j]], bufs[1], sem)

    def drain_gathers(j, bufs, sem):
        pltpu.make_async_copy(hx_hbm.at[u_idx.at[j]], bufs[0], sem).wait()
        pltpu.make_async_copy(hx_hbm.at[v_idx.at[j]], bufs[1], sem).wait()

    def fire_writes(j, bufs, sem):
        base = wid * LN_PER_W + j * LN_K
        for t in range(2):
            pltpu.async_copy(bufs[t], outs[t].at[pl.ds(base, LN_K)], sem)

    def drain_writes(j, bufs, sem):
        base = wid * LN_PER_W + j * LN_K
        for t in range(2):
            pltpu.make_async_copy(
                bufs[t], outs[t].at[pl.ds(base, LN_K)], sem).wait()

    def pair(p, c):
        j0 = 2 * p
        j1 = j0 + 1

        @pl.when(p > 0)
        def _():
            drain_writes(j0 - 2, bufs_a, swa)

        fire_gathers(j0, bufs_a, sga)

        @pl.when(p > 0)
        def _():
            drain_writes(j0 - 1, bufs_b, swb)

        fire_gathers(j1, bufs_b, sgb)
        drain_gathers(j0, bufs_a, sga)
        fire_writes(j0, bufs_a, swa)
        drain_gathers(j1, bufs_b, sgb)
        fire_writes(j1, bufs_b, swb)
        return c

    lax.fori_loop(0, LN_CHUNKS // 2, pair, 0)
    drain_writes(LN_CHUNKS - 2, bufs_a, swa)
    drain_writes(LN_CHUNKS - 1, bufs_b, swb)


_gathered_sds = jax.ShapeDtypeStruct((NLINK, DF), jnp.int32)
_ln_call = functools.partial(
    pl.kernel,
    mesh=plsc.VectorSubcoreMesh(core_axis_name="c", subcore_axis_name="s"),
    out_type=(_gathered_sds, _gathered_sds),
    scratch_types=(
        [pltpu.VMEM((LN_CHUNKS, LN_K), jnp.int32)] * 2
        + [pltpu.VMEM((LN_K, DF), jnp.int32)] * 4
        + [pltpu.SemaphoreType.DMA] * 4
    ),
)(_ln_body)


def _score_body(hxu, hxv, ef, wsn, wse, wg, out):
    i = pl.program_id(0)
    hxu_v = hxu[...]
    hxv_v = hxv[...]
    huv = _unpack(hxu_v[:, :DP]) * _unpack(hxv_v[:, :DP])
    xd = jnp.abs(_unpack(hxu_v[:, DP:]) - _unpack(hxv_v[:, DP:]))
    experts = (jnp.dot(huv, wsn[...], preferred_element_type=jnp.float32)
               + jnp.dot(ef[...], wse[...], preferred_element_type=jnp.float32))
    gl = jnp.dot(xd, wg[...], preferred_element_type=jnp.float32)
    col = lax.broadcasted_iota(jnp.int32, (SCORE_R, 8), 1)
    gl = jnp.where(col < 4, gl, -1e30)
    m = jnp.max(gl, axis=1, keepdims=True)
    p = jnp.exp(gl - m)
    gate = p / jnp.sum(p, axis=1, keepdims=True)
    logits = jnp.sum(experts * gate, axis=1, keepdims=True)
    r = lax.broadcasted_iota(jnp.int32, (SCORE_R, 1), 0)
    side_row = i * SCORE_R + r - jnp.where(i < SIDE_BLOCKS, 0, NSIDE)
    valid = side_row < NPOS
    tgt = jnp.where(i < SIDE_BLOCKS, 1.0, 0.0)
    bce = (jnp.maximum(logits, 0.0) - logits * tgt
           + jnp.log(1.0 + jnp.exp(-jnp.abs(logits))))
    s = jnp.sum(jnp.where(valid, bce, 0.0))

    @pl.when(i == 0)
    def _():
        out[...] = s.reshape(1, 1)

    @pl.when(i != 0)
    def _():
        out[...] += s.reshape(1, 1)


_score = pl.pallas_call(
    _score_body,
    grid=(SCORE_BLOCKS,),
    in_specs=[
        pl.BlockSpec((SCORE_R, DF), lambda i: (i, 0)),
        pl.BlockSpec((SCORE_R, DF), lambda i: (i, 0)),
        pl.BlockSpec((SCORE_R, DE), lambda i: (i, 0)),
        pl.BlockSpec((DF, 8), lambda i: (0, 0)),
        pl.BlockSpec((DE, 8), lambda i: (0, 0)),
        pl.BlockSpec((DF, 8), lambda i: (0, 0)),
    ],
    out_specs=pl.BlockSpec((1, 1), lambda i: (0, 0)),
    out_shape=jax.ShapeDtypeStruct((1, 1), jnp.float32),
)


def kernel(x, mp_link, pos_link, neg_link, pos_feats, neg_feats,
           W1, W_score, W_gate):
    f32 = jnp.float32
    msrc = mp_link[:, 0].reshape(NEDGE // MP_K, MP_K)
    mdst = mp_link[:, 1].reshape(NEDGE // MP_K, MP_K)
    pad_idx = (jnp.arange(NPAD, dtype=jnp.int32) * 13) % NN
    u_all = jnp.concatenate(
        [pos_link[:, 0], pad_idx, neg_link[:, 0], pad_idx]).reshape(
            NLINK // LN_K, LN_K)
    v_all = jnp.concatenate(
        [pos_link[:, 1], pad_idx, neg_link[:, 1], pad_idx]).reshape(
            NLINK // LN_K, LN_K)
    zpad = jnp.zeros((NPAD, DE), f32)
    ef = jnp.concatenate([pos_feats, zpad, neg_feats, zpad], axis=0)
    wsn = jnp.pad(W_score[:DF], ((0, 0), (0, 4)))
    wse = jnp.pad(W_score[DF:], ((0, 0), (0, 4)))
    wg = jnp.pad(W_gate, ((0, 0), (0, 4)))
    ztbl = jnp.zeros((NNP, DF), f32)

    xp = jnp.pad(x, ((0, NNP - NN), (0, 0)))
    xc = _prep(x)
    partials = _mp_call(xc, msrc, mdst, ztbl)
    hx = _node(partials.reshape(NC, NNP, DF), W1, xp)
    hxu, hxv = _ln_call(hx, u_all, v_all)
    s = _score(hxu, hxv, ef, wsn, wse, wg)
    return s[0, 0] * (1.0 / (NPOS + NNEG))


# speedup vs baseline: 4.6824x; 1.2210x over previous
import functools

import jax
import jax.numpy as jnp
from jax import lax
from jax.experimental import pallas as pl
from jax.experimental.pallas import tpu as pltpu
from jax.experimental.pallas import tpu_sc as plsc

NN = 10000
DF = 128
DE = 16
NEDGE = 320000
NPOS = 100000
NNEG = 100000
NSIDE = 102400
NLINK = 2 * NSIDE
NPAD = NSIDE - NPOS

NC = 2
NS = 16
NW = NC * NS

MP_K = 125
MP_CHUNKS = NEDGE // (NW * MP_K)
MP_GCH = 40
NNP = 10240
MP_ROWS_PER_TILE = NNP // NS

LN_K = 80
LN_CHUNKS = NLINK // (NW * LN_K)
LN_PER_W = NLINK // NW

SCORE_R = 4096
SCORE_BLOCKS = NLINK // SCORE_R
SIDE_BLOCKS = NSIDE // SCORE_R


DP = DF // 2


def _rne(u):
    return u + jnp.uint32(0x7FFF) + ((u >> 16) & jnp.uint32(1))


def _pack(t):
    u = lax.bitcast_convert_type(t, jnp.uint32)
    w = (_rne(u[:, :DP]) & jnp.uint32(0xFFFF0000)) | (_rne(u[:, DP:]) >> 16)
    return lax.bitcast_convert_type(w, jnp.int32)


def _unpack(p):
    u = lax.bitcast_convert_type(p, jnp.uint32)
    hi = lax.bitcast_convert_type(u & jnp.uint32(0xFFFF0000), jnp.float32)
    lo = lax.bitcast_convert_type(u << 16, jnp.float32)
    return jnp.concatenate([hi, lo], axis=1)


def _prep_body(x_ref, xc_ref):
    xv = x_ref[...]
    xc_ref[...] = xv - jnp.mean(xv, axis=0, keepdims=True)


_prep = pl.pallas_call(
    _prep_body,
    out_shape=jax.ShapeDtypeStruct((NN, DF), jnp.float32),
)


def _mp_body(xc_hbm, src_hbm, dst_hbm, zero_hbm, out_hbm,
             src_idx, dst_idx, rows_a, rows_b, agg_sh, ga, gb, sa, sb):
    cid = lax.axis_index("c")
    sid = lax.axis_index("s")
    wid = sid * NC + cid
    pltpu.sync_copy(zero_hbm.at[pl.ds(sid * MP_ROWS_PER_TILE, MP_ROWS_PER_TILE)],
                    agg_sh.at[pl.ds(sid * MP_ROWS_PER_TILE, MP_ROWS_PER_TILE)])
    plsc.subcore_barrier()

    def drain_scatter(j, rows, sem):
        pltpu.make_async_copy(rows, agg_sh.at[src_idx.at[j]], sem).wait()

    def group(g, c):
        base = wid * MP_CHUNKS + g * MP_GCH
        pltpu.sync_copy(src_hbm.at[pl.ds(base, MP_GCH)], src_idx)
        pltpu.sync_copy(dst_hbm.at[pl.ds(base, MP_GCH)], dst_idx)

        def pair(p, c2):
            j0 = 2 * p
            j1 = j0 + 1

            @pl.when(p > 0)
            def _():
                drain_scatter(j0 - 2, rows_a, sa)

            pltpu.async_copy(xc_hbm.at[dst_idx.at[j0]], rows_a, ga)

            @pl.when(p > 0)
            def _():
                drain_scatter(j0 - 1, rows_b, sb)

            pltpu.async_copy(xc_hbm.at[dst_idx.at[j1]], rows_b, gb)
            pltpu.make_async_copy(xc_hbm.at[dst_idx.at[j0]], rows_a, ga).wait()
            pltpu.async_copy(rows_a, agg_sh.at[src_idx.at[j0]], sa, add=True)
            pltpu.make_async_copy(xc_hbm.at[dst_idx.at[j1]], rows_b, gb).wait()
            pltpu.async_copy(rows_b, agg_sh.at[src_idx.at[j1]], sb, add=True)
            return c2

        lax.fori_loop(0, MP_GCH // 2, pair, 0)
        drain_scatter(MP_GCH - 2, rows_a, sa)
        drain_scatter(MP_GCH - 1, rows_b, sb)
        return c

    lax.fori_loop(0, MP_CHUNKS // MP_GCH, group, 0)
    plsc.subcore_barrier()
    base = cid * NNP + sid * MP_ROWS_PER_TILE
    pltpu.sync_copy(agg_sh.at[pl.ds(sid * MP_ROWS_PER_TILE, MP_ROWS_PER_TILE)],
                    out_hbm.at[pl.ds(base, MP_ROWS_PER_TILE)])


_mp_call = functools.partial(
    pl.kernel,
    mesh=plsc.VectorSubcoreMesh(core_axis_name="c", subcore_axis_name="s"),
    out_type=jax.ShapeDtypeStruct((NC * NNP, DF), jnp.float32),
    scratch_types=[
        pltpu.VMEM((MP_GCH, MP_K), jnp.int32),
        pltpu.VMEM((MP_GCH, MP_K), jnp.int32),
        pltpu.VMEM((MP_K, DF), jnp.float32),
        pltpu.VMEM((MP_K, DF), jnp.float32),
        pltpu.VMEM_SHARED((NNP, DF), jnp.float32),
        pltpu.SemaphoreType.DMA,
        pltpu.SemaphoreType.DMA,
        pltpu.SemaphoreType.DMA,
        pltpu.SemaphoreType.DMA,
    ],
)(_mp_body)


def _node_body(p_ref, w_ref, xp_ref, hx_ref):
    agg = p_ref[0] + p_ref[1]
    h = jnp.maximum(
        jnp.dot(agg, w_ref[...], preferred_element_type=jnp.float32), 0.0)
    hx_ref[...] = jnp.concatenate([_pack(h), _pack(xp_ref[...])], axis=1)


_node = pl.pallas_call(
    _node_body,
    grid=(10,),
    in_specs=[
        pl.BlockSpec((NC, NNP // 10, DF), lambda i: (0, i, 0)),
        pl.BlockSpec((DF, DF), lambda i: (0, 0)),
        pl.BlockSpec((NNP // 10, DF), lambda i: (i, 0)),
    ],
    out_specs=pl.BlockSpec((NNP // 10, DF), lambda i: (i, 0)),
    out_shape=jax.ShapeDtypeStruct((NNP, DF), jnp.int32),
)


def _ln_body(hx_hbm, u_hbm, v_hbm, ohxu, ohxv,
             u_idx, v_idx, a_u, a_v, b_u, b_v, sga, sgb, swa, swb):
    cid = lax.axis_index("c")
    sid = lax.axis_index("s")
    wid = sid * NC + cid
    pltpu.sync_copy(u_hbm.at[pl.ds(wid * LN_CHUNKS, LN_CHUNKS)], u_idx)
    pltpu.sync_copy(v_hbm.at[pl.ds(wid * LN_CHUNKS, LN_CHUNKS)], v_idx)

    bufs_a = (a_u, a_v)
    bufs_b = (b_u, b_v)
    outs = (ohxu, ohxv)

    def fire_gathers(j, bufs, sem):
        pltpu.async_copy(hx_hbm.at[u_idx.at[j]], bufs[0], sem)
        pltpu.async_copy(hx_hbm.at[v_idx.at[j]], bufs[1], sem)

    def drain_gathers(j, bufs, sem):
        pltpu.make_async_copy(hx_hbm.at[u_idx.at[j]], bufs[0], sem).wait()
        pltpu.make_async_copy(hx_hbm.at[v_idx.at[j]], bufs[1], sem).wait()

    def fire_writes(j, bufs, sem):
        base = wid * LN_PER_W + j * LN_K
        for t in range(2):
            pltpu.async_copy(bufs[t], outs[t].at[pl.ds(base, LN_K)], sem)

    def drain_writes(j, bufs, sem):
        base = wid * LN_PER_W + j * LN_K
        for t in range(2):
            pltpu.make_async_copy(
                bufs[t], outs[t].at[pl.ds(base, LN_K)], sem).wait()

    def pair(p, c):
        j0 = 2 * p
        j1 = j0 + 1

        @pl.when(p > 0)
        def _():
            drain_writes(j0 - 2, bufs_a, swa)

        fire_gathers(j0, bufs_a, sga)

        @pl.when(p > 0)
        def _():
            drain_writes(j0 - 1, bufs_b, swb)

        fire_gathers(j1, bufs_b, sgb)
        drain_gathers(j0, bufs_a, sga)
        fire_writes(j0, bufs_a, swa)
        drain_gathers(j1, bufs_b, sgb)
        fire_writes(j1, bufs_b, swb)
        return c

    lax.fori_loop(0, LN_CHUNKS // 2, pair, 0)
    drain_writes(LN_CHUNKS - 2, bufs_a, swa)
    drain_writes(LN_CHUNKS - 1, bufs_b, swb)


_gathered_sds = jax.ShapeDtypeStruct((NLINK, DF), jnp.int32)
_ln_call = functools.partial(
    pl.kernel,
    mesh=plsc.VectorSubcoreMesh(core_axis_name="c", subcore_axis_name="s"),
    out_type=(_gathered_sds, _gathered_sds),
    scratch_types=(
        [pltpu.VMEM((LN_CHUNKS, LN_K), jnp.int32)] * 2
        + [pltpu.VMEM((LN_K, DF), jnp.int32)] * 4
        + [pltpu.SemaphoreType.DMA] * 4
    ),
)(_ln_body)


def _score_body(hxu, hxv, pf, nf, wsn, wse, wg, out):
    i = pl.program_id(0)
    hxu_v = hxu[...]
    hxv_v = hxv[...]
    huv = _unpack(hxu_v[:, :DP]) * _unpack(hxv_v[:, :DP])
    xd = jnp.abs(_unpack(hxu_v[:, DP:]) - _unpack(hxv_v[:, DP:]))
    ef = jnp.where(i < SIDE_BLOCKS, pf[...], nf[...])
    experts = (jnp.dot(huv, wsn[...], preferred_element_type=jnp.float32)
               + jnp.dot(ef, wse[...], preferred_element_type=jnp.float32))
    gl = jnp.dot(xd, wg[...], preferred_element_type=jnp.float32)
    col = lax.broadcasted_iota(jnp.int32, (SCORE_R, 8), 1)
    gl = jnp.where(col < 4, gl, -1e30)
    m = jnp.max(gl, axis=1, keepdims=True)
    p = jnp.exp(gl - m)
    gate = p / jnp.sum(p, axis=1, keepdims=True)
    logits = jnp.sum(experts * gate, axis=1, keepdims=True)
    r = lax.broadcasted_iota(jnp.int32, (SCORE_R, 1), 0)
    side_row = i * SCORE_R + r - jnp.where(i < SIDE_BLOCKS, 0, NSIDE)
    valid = side_row < NPOS
    tgt = jnp.where(i < SIDE_BLOCKS, 1.0, 0.0)
    bce = (jnp.maximum(logits, 0.0) - logits * tgt
           + jnp.log(1.0 + jnp.exp(-jnp.abs(logits))))
    s = jnp.sum(jnp.where(valid, bce, 0.0))

    @pl.when(i == 0)
    def _():
        out[...] = s.reshape(1, 1)

    @pl.when(i != 0)
    def _():
        out[...] += s.reshape(1, 1)


_score = pl.pallas_call(
    _score_body,
    grid=(SCORE_BLOCKS,),
    in_specs=[
        pl.BlockSpec((SCORE_R, DF), lambda i: (i, 0)),
        pl.BlockSpec((SCORE_R, DF), lambda i: (i, 0)),
        pl.BlockSpec((SCORE_R, DE),
                     lambda i: (jnp.where(i < SIDE_BLOCKS, i, 0), 0)),
        pl.BlockSpec((SCORE_R, DE),
                     lambda i: (jnp.where(i < SIDE_BLOCKS, 0, i - SIDE_BLOCKS),
                                0)),
        pl.BlockSpec((DF, 8), lambda i: (0, 0)),
        pl.BlockSpec((DE, 8), lambda i: (0, 0)),
        pl.BlockSpec((DF, 8), lambda i: (0, 0)),
    ],
    out_specs=pl.BlockSpec((1, 1), lambda i: (0, 0)),
    out_shape=jax.ShapeDtypeStruct((1, 1), jnp.float32),
)


def kernel(x, mp_link, pos_link, neg_link, pos_feats, neg_feats,
           W1, W_score, W_gate):
    f32 = jnp.float32
    msrc = mp_link[:, 0].reshape(NEDGE // MP_K, MP_K)
    mdst = mp_link[:, 1].reshape(NEDGE // MP_K, MP_K)
    pad_idx = (jnp.arange(NPAD, dtype=jnp.int32) * 13) % NN
    u_all = jnp.concatenate(
        [pos_link[:, 0], pad_idx, neg_link[:, 0], pad_idx]).reshape(
            NLINK // LN_K, LN_K)
    v_all = jnp.concatenate(
        [pos_link[:, 1], pad_idx, neg_link[:, 1], pad_idx]).reshape(
            NLINK // LN_K, LN_K)
    pfp = jnp.pad(pos_feats, ((0, NPAD), (0, 0)))
    nfp = jnp.pad(neg_feats, ((0, NPAD), (0, 0)))
    wsn = jnp.pad(W_score[:DF], ((0, 0), (0, 4)))
    wse = jnp.pad(W_score[DF:], ((0, 0), (0, 4)))
    wg = jnp.pad(W_gate, ((0, 0), (0, 4)))
    ztbl = jnp.zeros((NNP, DF), f32)

    xp = jnp.pad(x, ((0, NNP - NN), (0, 0)))
    xc = _prep(x)
    partials = _mp_call(xc, msrc, mdst, ztbl)
    hx = _node(partials.reshape(NC, NNP, DF), W1, xp)
    hxu, hxv = _ln_call(hx, u_all, v_all)
    s = _score(hxu, hxv, pfp, nfp, wsn, wse, wg)
    return s[0, 0] * (1.0 / (NPOS + NNEG))


# speedup vs baseline: 5.2325x; 1.1175x over previous
import functools

import jax
import jax.numpy as jnp
from jax import lax
from jax.experimental import pallas as pl
from jax.experimental.pallas import tpu as pltpu
from jax.experimental.pallas import tpu_sc as plsc

NN = 10000
DF = 128
DE = 16
NEDGE = 320000
NPOS = 100000
NNEG = 100000
NSIDE = 102400
NLINK = 2 * NSIDE
NPAD = NSIDE - NPOS

NC = 2
NS = 16
NW = NC * NS

MP_K = 125
MP_CHUNKS = NEDGE // (NW * MP_K)
MP_GCH = 40
NNP = 10240
MP_ROWS_PER_TILE = NNP // NS

LN_K = 80
LN_CHUNKS = NLINK // (NW * LN_K)
LN_PER_W = NLINK // NW

SCORE_R = 4096
SCORE_BLOCKS = NLINK // SCORE_R
SIDE_BLOCKS = NSIDE // SCORE_R


DP = DF // 2


def _rne(u):
    return u + jnp.uint32(0x7FFF) + ((u >> 16) & jnp.uint32(1))


def _pack(t):
    u = lax.bitcast_convert_type(t, jnp.uint32)
    w = (_rne(u[:, :DP]) & jnp.uint32(0xFFFF0000)) | (_rne(u[:, DP:]) >> 16)
    return lax.bitcast_convert_type(w, jnp.int32)


def _unpack(p):
    u = lax.bitcast_convert_type(p, jnp.uint32)
    hi = lax.bitcast_convert_type(u & jnp.uint32(0xFFFF0000), jnp.float32)
    lo = lax.bitcast_convert_type(u << 16, jnp.float32)
    return jnp.concatenate([hi, lo], axis=1)


def _prep_body(x_ref, xc_ref):
    xv = x_ref[...]
    xc_ref[...] = xv - jnp.mean(xv, axis=0, keepdims=True)


_prep = pl.pallas_call(
    _prep_body,
    out_shape=jax.ShapeDtypeStruct((NN, DF), jnp.float32),
)


def _mp_body(xc_hbm, src_hbm, dst_hbm, zero_hbm, out_hbm,
             src_idx, dst_idx, rows_a, rows_b, agg_sh, ga, gb, sa, sb):
    cid = lax.axis_index("c")
    sid = lax.axis_index("s")
    wid = sid * NC + cid
    pltpu.sync_copy(zero_hbm.at[pl.ds(sid * MP_ROWS_PER_TILE, MP_ROWS_PER_TILE)],
                    agg_sh.at[pl.ds(sid * MP_ROWS_PER_TILE, MP_ROWS_PER_TILE)])
    plsc.subcore_barrier()

    def drain_scatter(j, rows, sem):
        pltpu.make_async_copy(rows, agg_sh.at[src_idx.at[j]], sem).wait()

    def group(g, c):
        base = wid * MP_CHUNKS + g * MP_GCH
        pltpu.sync_copy(src_hbm.at[pl.ds(base, MP_GCH)], src_idx)
        pltpu.sync_copy(dst_hbm.at[pl.ds(base, MP_GCH)], dst_idx)

        def pair(p, c2):
            j0 = 2 * p
            j1 = j0 + 1

            @pl.when(p > 0)
            def _():
                drain_scatter(j0 - 2, rows_a, sa)

            pltpu.async_copy(xc_hbm.at[dst_idx.at[j0]], rows_a, ga)

            @pl.when(p > 0)
            def _():
                drain_scatter(j0 - 1, rows_b, sb)

            pltpu.async_copy(xc_hbm.at[dst_idx.at[j1]], rows_b, gb)
            pltpu.make_async_copy(xc_hbm.at[dst_idx.at[j0]], rows_a, ga).wait()
            pltpu.async_copy(rows_a, agg_sh.at[src_idx.at[j0]], sa, add=True)
            pltpu.make_async_copy(xc_hbm.at[dst_idx.at[j1]], rows_b, gb).wait()
            pltpu.async_copy(rows_b, agg_sh.at[src_idx.at[j1]], sb, add=True)
            return c2

        lax.fori_loop(0, MP_GCH // 2, pair, 0)
        drain_scatter(MP_GCH - 2, rows_a, sa)
        drain_scatter(MP_GCH - 1, rows_b, sb)
        return c

    lax.fori_loop(0, MP_CHUNKS // MP_GCH, group, 0)
    plsc.subcore_barrier()
    base = cid * NNP + sid * MP_ROWS_PER_TILE
    pltpu.sync_copy(agg_sh.at[pl.ds(sid * MP_ROWS_PER_TILE, MP_ROWS_PER_TILE)],
                    out_hbm.at[pl.ds(base, MP_ROWS_PER_TILE)])


_mp_call = functools.partial(
    pl.kernel,
    mesh=plsc.VectorSubcoreMesh(core_axis_name="c", subcore_axis_name="s"),
    out_type=jax.ShapeDtypeStruct((NC * NNP, DF), jnp.float32),
    scratch_types=[
        pltpu.VMEM((MP_GCH, MP_K), jnp.int32),
        pltpu.VMEM((MP_GCH, MP_K), jnp.int32),
        pltpu.VMEM((MP_K, DF), jnp.float32),
        pltpu.VMEM((MP_K, DF), jnp.float32),
        pltpu.VMEM_SHARED((NNP, DF), jnp.float32),
        pltpu.SemaphoreType.DMA,
        pltpu.SemaphoreType.DMA,
        pltpu.SemaphoreType.DMA,
        pltpu.SemaphoreType.DMA,
    ],
)(_mp_body)


def _node_body(p_ref, w_ref, xp_ref, hx_ref):
    agg = p_ref[0] + p_ref[1]
    h = jnp.maximum(
        jnp.dot(agg, w_ref[...], preferred_element_type=jnp.float32), 0.0)
    hx_ref[...] = jnp.concatenate([_pack(h), _pack(xp_ref[...])], axis=1)


_node = pl.pallas_call(
    _node_body,
    grid=(10,),
    in_specs=[
        pl.BlockSpec((NC, NNP // 10, DF), lambda i: (0, i, 0)),
        pl.BlockSpec((DF, DF), lambda i: (0, 0)),
        pl.BlockSpec((NNP // 10, DF), lambda i: (i, 0)),
    ],
    out_specs=pl.BlockSpec((NNP // 10, DF), lambda i: (i, 0)),
    out_shape=jax.ShapeDtypeStruct((NNP, DF), jnp.int32),
)


def _ln_body(hx_hbm, u_hbm, v_hbm, ohxu, ohxv,
             u_idx, v_idx, a_u, a_v, b_u, b_v, sga, sgb, swa, swb):
    cid = lax.axis_index("c")
    sid = lax.axis_index("s")
    wid = sid * NC + cid
    pltpu.sync_copy(u_hbm.at[pl.ds(wid * LN_CHUNKS, LN_CHUNKS)], u_idx)
    pltpu.sync_copy(v_hbm.at[pl.ds(wid * LN_CHUNKS, LN_CHUNKS)], v_idx)

    bufs_a = (a_u, a_v)
    bufs_b = (b_u, b_v)
    outs = (ohxu, ohxv)

    def fire_gathers(j, bufs, sem):
        pltpu.async_copy(hx_hbm.at[u_idx.at[j]], bufs[0], sem)
        pltpu.async_copy(hx_hbm.at[v_idx.at[j]], bufs[1], sem)

    def drain_gathers(j, bufs, sem):
        pltpu.make_async_copy(hx_hbm.at[u_idx.at[j]], bufs[0], sem).wait()
        pltpu.make_async_copy(hx_hbm.at[v_idx.at[j]], bufs[1], sem).wait()

    def fire_writes(j, bufs, sem):
        base = wid * LN_PER_W + j * LN_K
        for t in range(2):
            pltpu.async_copy(bufs[t], outs[t].at[pl.ds(base, LN_K)], sem)

    def drain_writes(j, bufs, sem):
        base = wid * LN_PER_W + j * LN_K
        for t in range(2):
            pltpu.make_async_copy(
                bufs[t], outs[t].at[pl.ds(base, LN_K)], sem).wait()

    def pair(p, c):
        j0 = 2 * p
        j1 = j0 + 1

        @pl.when(p > 0)
        def _():
            drain_writes(j0 - 2, bufs_a, swa)

        fire_gathers(j0, bufs_a, sga)

        @pl.when(p > 0)
        def _():
            drain_writes(j0 - 1, bufs_b, swb)

        fire_gathers(j1, bufs_b, sgb)
        drain_gathers(j0, bufs_a, sga)
        fire_writes(j0, bufs_a, swa)
        drain_gathers(j1, bufs_b, sgb)
        fire_writes(j1, bufs_b, swb)
        return c

    lax.fori_loop(0, LN_CHUNKS // 2, pair, 0)
    drain_writes(LN_CHUNKS - 2, bufs_a, swa)
    drain_writes(LN_CHUNKS - 1, bufs_b, swb)


_gathered_sds = jax.ShapeDtypeStruct((NLINK, DF), jnp.int32)
_ln_call = functools.partial(
    pl.kernel,
    mesh=plsc.VectorSubcoreMesh(core_axis_name="c", subcore_axis_name="s"),
    out_type=(_gathered_sds, _gathered_sds),
    scratch_types=(
        [pltpu.VMEM((LN_CHUNKS, LN_K), jnp.int32)] * 2
        + [pltpu.VMEM((LN_K, DF), jnp.int32)] * 4
        + [pltpu.SemaphoreType.DMA] * 4
    ),
)(_ln_body)


def _score_body(hxu, hxv, pf, nf, wsn, wse, wg, out):
    i = pl.program_id(0)
    hxu_v = hxu[...]
    hxv_v = hxv[...]
    huv = _unpack(hxu_v[:, :DP]) * _unpack(hxv_v[:, :DP])
    xd = jnp.abs(_unpack(hxu_v[:, DP:]) - _unpack(hxv_v[:, DP:]))
    ef = jnp.where(i < SIDE_BLOCKS, pf[...], nf[...])
    experts = (jnp.dot(huv, wsn[...], preferred_element_type=jnp.float32)
               + jnp.dot(ef, wse[...], preferred_element_type=jnp.float32))
    gl = jnp.dot(xd, wg[...], preferred_element_type=jnp.float32)
    eT = experts.T
    gT = gl.T
    erow = lax.broadcasted_iota(jnp.int32, (8, SCORE_R), 0)
    gT = jnp.where(erow < 4, gT, -1e30)
    m = jnp.max(gT, axis=0, keepdims=True)
    p = jnp.exp(gT - m)
    gate = p / jnp.sum(p, axis=0, keepdims=True)
    logits = jnp.sum(eT * gate, axis=0, keepdims=True)
    r = lax.broadcasted_iota(jnp.int32, (1, SCORE_R), 1)
    side_row = i * SCORE_R + r - jnp.where(i < SIDE_BLOCKS, 0, NSIDE)
    valid = side_row < NPOS
    tgt = jnp.where(i < SIDE_BLOCKS, 1.0, 0.0)
    bce = (jnp.maximum(logits, 0.0) - logits * tgt
           + jnp.log(1.0 + jnp.exp(-jnp.abs(logits))))
    s = jnp.sum(jnp.where(valid, bce, 0.0))

    @pl.when(i == 0)
    def _():
        out[...] = s.reshape(1, 1)

    @pl.when(i != 0)
    def _():
        out[...] += s.reshape(1, 1)


_score = pl.pallas_call(
    _score_body,
    grid=(SCORE_BLOCKS,),
    in_specs=[
        pl.BlockSpec((SCORE_R, DF), lambda i: (i, 0)),
        pl.BlockSpec((SCORE_R, DF), lambda i: (i, 0)),
        pl.BlockSpec((SCORE_R, DE),
                     lambda i: (jnp.where(i < SIDE_BLOCKS, i, 0), 0)),
        pl.BlockSpec((SCORE_R, DE),
                     lambda i: (jnp.where(i < SIDE_BLOCKS, 0, i - SIDE_BLOCKS),
                                0)),
        pl.BlockSpec((DF, 8), lambda i: (0, 0)),
        pl.BlockSpec((DE, 8), lambda i: (0, 0)),
        pl.BlockSpec((DF, 8), lambda i: (0, 0)),
    ],
    out_specs=pl.BlockSpec((1, 1), lambda i: (0, 0)),
    out_shape=jax.ShapeDtypeStruct((1, 1), jnp.float32),
)


def kernel(x, mp_link, pos_link, neg_link, pos_feats, neg_feats,
           W1, W_score, W_gate):
    f32 = jnp.float32
    msrc = mp_link[:, 0].reshape(NEDGE // MP_K, MP_K)
    mdst = mp_link[:, 1].reshape(NEDGE // MP_K, MP_K)
    pad_idx = (jnp.arange(NPAD, dtype=jnp.int32) * 13) % NN
    u_all = jnp.concatenate(
        [pos_link[:, 0], pad_idx, neg_link[:, 0], pad_idx]).reshape(
            NLINK // LN_K, LN_K)
    v_all = jnp.concatenate(
        [pos_link[:, 1], pad_idx, neg_link[:, 1], pad_idx]).reshape(
            NLINK // LN_K, LN_K)
    pfp = jnp.pad(pos_feats, ((0, NPAD), (0, 0)))
    nfp = jnp.pad(neg_feats, ((0, NPAD), (0, 0)))
    wsn = jnp.pad(W_score[:DF], ((0, 0), (0, 4)))
    wse = jnp.pad(W_score[DF:], ((0, 0), (0, 4)))
    wg = jnp.pad(W_gate, ((0, 0), (0, 4)))
    ztbl = jnp.zeros((NNP, DF), f32)

    xp = jnp.pad(x, ((0, NNP - NN), (0, 0)))
    xc = _prep(x)
    partials = _mp_call(xc, msrc, mdst, ztbl)
    hx = _node(partials.reshape(NC, NNP, DF), W1, xp)
    hxu, hxv = _ln_call(hx, u_all, v_all)
    s = _score(hxu, hxv, pfp, nfp, wsn, wse, wg)
    return s[0, 0] * (1.0 / (NPOS + NNEG))


# speedup vs baseline: 5.2438x; 1.0022x over previous
import functools

import jax
import jax.numpy as jnp
from jax import lax
from jax.experimental import pallas as pl
from jax.experimental.pallas import tpu as pltpu
from jax.experimental.pallas import tpu_sc as plsc

NN = 10000
DF = 128
DE = 16
NEDGE = 320000
NPOS = 100000
NNEG = 100000
NSIDE = 102400
NLINK = 2 * NSIDE
NPAD = NSIDE - NPOS

NC = 2
NS = 16
NW = NC * NS

MP_K = 125
MP_CHUNKS = NEDGE // (NW * MP_K)
MP_GCH = 40
NNP = 10240
MP_ROWS_PER_TILE = NNP // NS

LN_K = 128
LN_CHUNKS = NLINK // (NW * LN_K)
LN_PER_W = NLINK // NW

SCORE_R = 4096
SCORE_BLOCKS = NLINK // SCORE_R
SIDE_BLOCKS = NSIDE // SCORE_R


DP = DF // 2


def _rne(u):
    return u + jnp.uint32(0x7FFF) + ((u >> 16) & jnp.uint32(1))


def _pack(t):
    u = lax.bitcast_convert_type(t, jnp.uint32)
    w = (_rne(u[:, :DP]) & jnp.uint32(0xFFFF0000)) | (_rne(u[:, DP:]) >> 16)
    return lax.bitcast_convert_type(w, jnp.int32)


def _unpack(p):
    u = lax.bitcast_convert_type(p, jnp.uint32)
    hi = lax.bitcast_convert_type(u & jnp.uint32(0xFFFF0000), jnp.float32)
    lo = lax.bitcast_convert_type(u << 16, jnp.float32)
    return jnp.concatenate([hi, lo], axis=1)


def _prep_body(x_ref, xc_ref):
    xv = x_ref[...]
    xc_ref[...] = xv - jnp.mean(xv, axis=0, keepdims=True)


_prep = pl.pallas_call(
    _prep_body,
    out_shape=jax.ShapeDtypeStruct((NN, DF), jnp.float32),
)


def _mp_body(xc_hbm, src_hbm, dst_hbm, zero_hbm, out_hbm,
             src_idx, dst_idx, rows_a, rows_b, agg_sh, ga, gb, sa, sb):
    cid = lax.axis_index("c")
    sid = lax.axis_index("s")
    wid = sid * NC + cid
    pltpu.sync_copy(zero_hbm.at[pl.ds(sid * MP_ROWS_PER_TILE, MP_ROWS_PER_TILE)],
                    agg_sh.at[pl.ds(sid * MP_ROWS_PER_TILE, MP_ROWS_PER_TILE)])
    plsc.subcore_barrier()

    def drain_scatter(j, rows, sem):
        pltpu.make_async_copy(rows, agg_sh.at[src_idx.at[j]], sem).wait()

    def group(g, c):
        base = wid * MP_CHUNKS + g * MP_GCH
        pltpu.sync_copy(src_hbm.at[pl.ds(base, MP_GCH)], src_idx)
        pltpu.sync_copy(dst_hbm.at[pl.ds(base, MP_GCH)], dst_idx)

        def pair(p, c2):
            j0 = 2 * p
            j1 = j0 + 1

            @pl.when(p > 0)
            def _():
                drain_scatter(j0 - 2, rows_a, sa)

            pltpu.async_copy(xc_hbm.at[dst_idx.at[j0]], rows_a, ga)

            @pl.when(p > 0)
            def _():
                drain_scatter(j0 - 1, rows_b, sb)

            pltpu.async_copy(xc_hbm.at[dst_idx.at[j1]], rows_b, gb)
            pltpu.make_async_copy(xc_hbm.at[dst_idx.at[j0]], rows_a, ga).wait()
            pltpu.async_copy(rows_a, agg_sh.at[src_idx.at[j0]], sa, add=True)
            pltpu.make_async_copy(xc_hbm.at[dst_idx.at[j1]], rows_b, gb).wait()
            pltpu.async_copy(rows_b, agg_sh.at[src_idx.at[j1]], sb, add=True)
            return c2

        lax.fori_loop(0, MP_GCH // 2, pair, 0)
        drain_scatter(MP_GCH - 2, rows_a, sa)
        drain_scatter(MP_GCH - 1, rows_b, sb)
        return c

    lax.fori_loop(0, MP_CHUNKS // MP_GCH, group, 0)
    plsc.subcore_barrier()
    base = cid * NNP + sid * MP_ROWS_PER_TILE
    pltpu.sync_copy(agg_sh.at[pl.ds(sid * MP_ROWS_PER_TILE, MP_ROWS_PER_TILE)],
                    out_hbm.at[pl.ds(base, MP_ROWS_PER_TILE)])


_mp_call = functools.partial(
    pl.kernel,
    mesh=plsc.VectorSubcoreMesh(core_axis_name="c", subcore_axis_name="s"),
    out_type=jax.ShapeDtypeStruct((NC * NNP, DF), jnp.float32),
    scratch_types=[
        pltpu.VMEM((MP_GCH, MP_K), jnp.int32),
        pltpu.VMEM((MP_GCH, MP_K), jnp.int32),
        pltpu.VMEM((MP_K, DF), jnp.float32),
        pltpu.VMEM((MP_K, DF), jnp.float32),
        pltpu.VMEM_SHARED((NNP, DF), jnp.float32),
        pltpu.SemaphoreType.DMA,
        pltpu.SemaphoreType.DMA,
        pltpu.SemaphoreType.DMA,
        pltpu.SemaphoreType.DMA,
    ],
)(_mp_body)


def _node_body(p_ref, w_ref, xp_ref, hx_ref):
    agg = p_ref[0] + p_ref[1]
    h = jnp.maximum(
        jnp.dot(agg, w_ref[...], preferred_element_type=jnp.float32), 0.0)
    hx_ref[...] = jnp.concatenate([_pack(h), _pack(xp_ref[...])], axis=1)


_node = pl.pallas_call(
    _node_body,
    grid=(10,),
    in_specs=[
        pl.BlockSpec((NC, NNP // 10, DF), lambda i: (0, i, 0)),
        pl.BlockSpec((DF, DF), lambda i: (0, 0)),
        pl.BlockSpec((NNP // 10, DF), lambda i: (i, 0)),
    ],
    out_specs=pl.BlockSpec((NNP // 10, DF), lambda i: (i, 0)),
    out_shape=jax.ShapeDtypeStruct((NNP, DF), jnp.int32),
)


def _ln_body(hx_hbm, u_hbm, v_hbm, ohxu, ohxv,
             u_idx, v_idx, a_u, a_v, b_u, b_v, sga, sgb, swa, swb):
    cid = lax.axis_index("c")
    sid = lax.axis_index("s")
    wid = sid * NC + cid
    pltpu.sync_copy(u_hbm.at[wid], u_idx)
    pltpu.sync_copy(v_hbm.at[wid], v_idx)

    bufs_a = (a_u, a_v)
    bufs_b = (b_u, b_v)
    outs = (ohxu, ohxv)

    def fire_gathers(j, bufs, sem):
        pltpu.async_copy(hx_hbm.at[u_idx.at[j]], bufs[0], sem)
        pltpu.async_copy(hx_hbm.at[v_idx.at[j]], bufs[1], sem)

    def drain_gathers(j, bufs, sem):
        pltpu.make_async_copy(hx_hbm.at[u_idx.at[j]], bufs[0], sem).wait()
        pltpu.make_async_copy(hx_hbm.at[v_idx.at[j]], bufs[1], sem).wait()

    def fire_writes(j, bufs, sem):
        base = wid * LN_PER_W + j * LN_K
        for t in range(2):
            pltpu.async_copy(bufs[t], outs[t].at[pl.ds(base, LN_K)], sem)

    def drain_writes(j, bufs, sem):
        base = wid * LN_PER_W + j * LN_K
        for t in range(2):
            pltpu.make_async_copy(
                bufs[t], outs[t].at[pl.ds(base, LN_K)], sem).wait()

    def pair(p, c):
        j0 = 2 * p
        j1 = j0 + 1

        @pl.when(p > 0)
        def _():
            drain_writes(j0 - 2, bufs_a, swa)

        fire_gathers(j0, bufs_a, sga)

        @pl.when(p > 0)
        def _():
            drain_writes(j0 - 1, bufs_b, swb)

        fire_gathers(j1, bufs_b, sgb)
        drain_gathers(j0, bufs_a, sga)
        fire_writes(j0, bufs_a, swa)
        drain_gathers(j1, bufs_b, sgb)
        fire_writes(j1, bufs_b, swb)
        return c

    lax.fori_loop(0, LN_CHUNKS // 2, pair, 0)
    drain_writes(LN_CHUNKS - 2, bufs_a, swa)
    drain_writes(LN_CHUNKS - 1, bufs_b, swb)


_gathered_sds = jax.ShapeDtypeStruct((NLINK, DF), jnp.int32)
_ln_call = functools.partial(
    pl.kernel,
    mesh=plsc.VectorSubcoreMesh(core_axis_name="c", subcore_axis_name="s"),
    out_type=(_gathered_sds, _gathered_sds),
    scratch_types=(
        [pltpu.VMEM((LN_CHUNKS, LN_K), jnp.int32)] * 2
        + [pltpu.VMEM((LN_K, DF), jnp.int32)] * 4
        + [pltpu.SemaphoreType.DMA] * 4
    ),
)(_ln_body)


def _score_body(hxu, hxv, pf, nf, wsn, wse, wg, out):
    i = pl.program_id(0)
    hxu_v = hxu[...]
    hxv_v = hxv[...]
    huv = _unpack(hxu_v[:, :DP]) * _unpack(hxv_v[:, :DP])
    xd = jnp.abs(_unpack(hxu_v[:, DP:]) - _unpack(hxv_v[:, DP:]))
    ef = jnp.where(i < SIDE_BLOCKS, pf[...], nf[...])
    experts = (jnp.dot(huv, wsn[...], preferred_element_type=jnp.float32)
               + jnp.dot(ef, wse[...], preferred_element_type=jnp.float32))
    gl = jnp.dot(xd, wg[...], preferred_element_type=jnp.float32)
    eT = experts.T
    gT = gl.T
    erow = lax.broadcasted_iota(jnp.int32, (8, SCORE_R), 0)
    gT = jnp.where(erow < 4, gT, -1e30)
    m = jnp.max(gT, axis=0, keepdims=True)
    p = jnp.exp(gT - m)
    gate = p / jnp.sum(p, axis=0, keepdims=True)
    logits = jnp.sum(eT * gate, axis=0, keepdims=True)
    r = lax.broadcasted_iota(jnp.int32, (1, SCORE_R), 1)
    side_row = i * SCORE_R + r - jnp.where(i < SIDE_BLOCKS, 0, NSIDE)
    valid = side_row < NPOS
    tgt = jnp.where(i < SIDE_BLOCKS, 1.0, 0.0)
    bce = (jnp.maximum(logits, 0.0) - logits * tgt
           + jnp.log(1.0 + jnp.exp(-jnp.abs(logits))))
    s = jnp.sum(jnp.where(valid, bce, 0.0))

    @pl.when(i == 0)
    def _():
        out[...] = s.reshape(1, 1)

    @pl.when(i != 0)
    def _():
        out[...] += s.reshape(1, 1)


_score = pl.pallas_call(
    _score_body,
    grid=(SCORE_BLOCKS,),
    in_specs=[
        pl.BlockSpec((SCORE_R, DF), lambda i: (i, 0)),
        pl.BlockSpec((SCORE_R, DF), lambda i: (i, 0)),
        pl.BlockSpec((SCORE_R, DE),
                     lambda i: (jnp.where(i < SIDE_BLOCKS, i, 0), 0)),
        pl.BlockSpec((SCORE_R, DE),
                     lambda i: (jnp.where(i < SIDE_BLOCKS, 0, i - SIDE_BLOCKS),
                                0)),
        pl.BlockSpec((DF, 8), lambda i: (0, 0)),
        pl.BlockSpec((DE, 8), lambda i: (0, 0)),
        pl.BlockSpec((DF, 8), lambda i: (0, 0)),
    ],
    out_specs=pl.BlockSpec((1, 1), lambda i: (0, 0)),
    out_shape=jax.ShapeDtypeStruct((1, 1), jnp.float32),
)


def kernel(x, mp_link, pos_link, neg_link, pos_feats, neg_feats,
           W1, W_score, W_gate):
    f32 = jnp.float32
    msrc = mp_link[:, 0].reshape(NEDGE // MP_K, MP_K)
    mdst = mp_link[:, 1].reshape(NEDGE // MP_K, MP_K)
    pad_idx = (jnp.arange(NPAD, dtype=jnp.int32) * 13) % NN
    u_all = jnp.concatenate(
        [pos_link[:, 0], pad_idx, neg_link[:, 0], pad_idx]).reshape(
            NW, LN_CHUNKS, LN_K)
    v_all = jnp.concatenate(
        [pos_link[:, 1], pad_idx, neg_link[:, 1], pad_idx]).reshape(
            NW, LN_CHUNKS, LN_K)
    pfp = jnp.pad(pos_feats, ((0, NPAD), (0, 0)))
    nfp = jnp.pad(neg_feats, ((0, NPAD), (0, 0)))
    wsn = jnp.pad(W_score[:DF], ((0, 0), (0, 4)))
    wse = jnp.pad(W_score[DF:], ((0, 0), (0, 4)))
    wg = jnp.pad(W_gate, ((0, 0), (0, 4)))
    ztbl = jnp.zeros((NNP, DF), f32)

    xp = jnp.pad(x, ((0, NNP - NN), (0, 0)))
    xc = _prep(x)
    partials = _mp_call(xc, msrc, mdst, ztbl)
    hx = _node(partials.reshape(NC, NNP, DF), W1, xp)
    hxu, hxv = _ln_call(hx, u_all, v_all)
    s = _score(hxu, hxv, pfp, nfp, wsn, wse, wg)
    return s[0, 0] * (1.0 / (NPOS + NNEG))
